# baseline (speedup 1.0000x reference)
import functools

import jax
import jax.numpy as jnp
from jax import lax
from jax.experimental import pallas as pl
from jax.experimental.pallas import tpu as pltpu
from jax.experimental.pallas import tpu_sc as plsc

EPS = 1e-10
MAX_NORM = 1.0 - 1e-5

NC = 2
NS = 16
LANES = 16

_MESH = dict(core_axis_name="c", subcore_axis_name="s", num_cores=NC,
             num_subcores=NS)


def _rownorm(x):
    return jnp.sqrt(jnp.sum(x * x, axis=-1, keepdims=True))


def _project(x):
    norm = jnp.maximum(_rownorm(x), EPS)
    scale = jnp.where(norm > MAX_NORM, MAX_NORM / norm, 1.0)
    return x * scale


def _mobius_add(x, y):
    x2 = jnp.sum(x * x, axis=-1, keepdims=True)
    y2 = jnp.sum(y * y, axis=-1, keepdims=True)
    xy = jnp.sum(x * y, axis=-1, keepdims=True)
    num = (1.0 + 2.0 * xy + y2) * x + (1.0 - x2) * y
    den = jnp.maximum(1.0 + 2.0 * xy + x2 * y2, EPS)
    return num / den


def _edge_coeffs(h2, t2, r2, d_ht, d_hr, d_tr):
    un_h = jnp.maximum(jnp.sqrt(h2), EPS)
    f = jnp.tanh(un_h) / un_h
    nh = jnp.sqrt(f * f * h2)
    f = f * jnp.where(nh > MAX_NORM, MAX_NORM / nh, 1.0)
    hh2 = f * f * h2
    two_over_lam = jnp.maximum(1.0 - hh2, EPS)
    hl = 1.0 / two_over_lam

    def sec(v2):
        un = jnp.maximum(jnp.sqrt(v2), EPS)
        return jnp.tanh(hl * un) / un

    def mobius(x2, y2, xy):
        den = jnp.maximum(1.0 + 2.0 * xy + x2 * y2, EPS)
        inv = 1.0 / den
        return (1.0 + 2.0 * xy + y2) * inv, (1.0 - x2) * inv

    def proj2(n2):
        n = jnp.sqrt(n2)
        return jnp.where(n > MAX_NORM, MAX_NORM / n, 1.0)

    g_t = sec(t2)
    st2 = g_t * g_t * t2
    A, B = mobius(hh2, st2, f * g_t * d_ht)
    a1, b1 = A * f, B * g_t
    ht2 = a1 * a1 * h2 + b1 * b1 * t2 + 2.0 * a1 * b1 * d_ht
    s = proj2(ht2)
    a1, b1, ht2 = a1 * s, b1 * s, ht2 * s * s

    g_r = sec(r2)
    sr2 = g_r * g_r * r2
    A, B = mobius(hh2, sr2, f * g_r * d_hr)
    a2, c2 = A * f, B * g_r
    hr2 = a2 * a2 * h2 + c2 * c2 * r2 + 2.0 * a2 * c2 * d_hr
    s = proj2(hr2)
    a2, c2, hr2 = a2 * s, c2 * s, hr2 * s * s

    xy = a1 * a2 * h2 + a1 * c2 * d_hr + b1 * a2 * d_ht + b1 * c2 * d_tr
    A, B = mobius(ht2, hr2, xy)
    al, be, ga = A * a1 + B * a2, A * b1, B * c2
    m2 = (al * al * h2 + be * be * t2 + ga * ga * r2
          + 2.0 * (al * be * d_ht + al * ga * d_hr + be * ga * d_tr))
    s = proj2(m2)
    al, be, ga, m2 = al * s, be * s, ga * s, m2 * s * s

    xy = -f * (al * h2 + be * d_ht + ga * d_hr)
    A, B = mobius(hh2, m2, xy)
    p, q, w = B * al - A * f, B * be, B * ga
    sub2 = (p * p * h2 + q * q * t2 + w * w * r2
            + 2.0 * (p * q * d_ht + p * w * d_hr + q * w * d_tr))

    sn = jnp.maximum(jnp.sqrt(sub2), EPS)
    snc = jnp.minimum(sn, MAX_NORM)
    artanh = 0.5 * (jnp.log1p(snc) - jnp.log1p(-snc))
    k0 = two_over_lam * artanh / sn
    return k0 * p, k0 * q, k0 * w


def _edge_math_block(head_emb, tail_emb, rel_emb):
    dot = lambda x, y: jnp.sum(x * y, axis=-1, keepdims=True)
    h2 = dot(head_emb, head_emb)
    t2 = dot(tail_emb, tail_emb)
    r2 = dot(rel_emb, rel_emb)
    d_ht = dot(head_emb, tail_emb)
    d_hr = dot(head_emb, rel_emb)
    d_tr = dot(tail_emb, rel_emb)
    P, Q, W = _edge_coeffs(h2, t2, r2, d_ht, d_hr, d_tr)
    return P * head_emb + Q * tail_emb + W * rel_emb


def _edge_kernel(head_ref, tail_ref, et_ref, rw_ref, out_ref):
    rel = et_ref[0, 0, :] - 1
    oh = (rel[:, None] == lax.iota(jnp.int32, 16)[None, :]).astype(jnp.float32)
    rel_emb = jnp.dot(oh, rw_ref[...], preferred_element_type=jnp.float32)
    out_ref[...] = _edge_math_block(head_ref[...], tail_ref[...], rel_emb)


def _pick_block(e, target=2000):
    best = 8
    for b in range(8, min(e, 4096) + 1, 8):
        if e % b == 0 and abs(b - target) < abs(best - target):
            best = b
    return best


def _edge_transform(head_emb, tail_emb, edge_type, rw_pad):
    e, c = head_emb.shape
    be = _pick_block(e)
    nb = e // be
    et3 = edge_type.reshape(nb, 1, be)
    return pl.pallas_call(
        _edge_kernel,
        grid=(nb,),
        in_specs=[
            pl.BlockSpec((be, c), lambda i: (i, 0)),
            pl.BlockSpec((be, c), lambda i: (i, 0)),
            pl.BlockSpec((1, 1, be), lambda i: (i, 0, 0)),
            pl.BlockSpec((16, c), lambda i: (0, 0)),
        ],
        out_specs=pl.BlockSpec((be, c), lambda i: (i, 0)),
        out_shape=jax.ShapeDtypeStruct((e, c), jnp.float32),
    )(head_emb, tail_emb, et3, rw_pad)


def _wid():
    return lax.axis_index("s") * NC + lax.axis_index("c")


def _sc_gather2(table, idx_a, idx_b):
    e = idx_a.shape[0]
    c = table.shape[1]
    k = 1000
    per_w = e // (NC * NS)
    n_ch = per_w // k
    assert per_w % k == 0

    @functools.partial(
        pl.kernel,
        mesh=plsc.VectorSubcoreMesh(**_MESH),
        compiler_params=pltpu.CompilerParams(use_tc_tiling_on_sc=False),
        out_type=[jax.ShapeDtypeStruct((e, c), jnp.float32),
                  jax.ShapeDtypeStruct((e, c), jnp.float32)],
        scratch_types=[
            pltpu.VMEM((k,), jnp.int32),
            pltpu.VMEM((k, c), jnp.float32),
            pltpu.SemaphoreType.DMA,
        ],
    )
    def body(table_h, ia_h, ib_h, oa_h, ob_h, idx_v, rows_v, sem):
        base0 = _wid() * per_w

        def step(j, _):
            base = base0 + j * k
            pltpu.sync_copy(ia_h.at[pl.ds(base, k)], idx_v)
            pltpu.async_copy(table_h.at[idx_v], rows_v, sem).wait()
            pltpu.sync_copy(rows_v, oa_h.at[pl.ds(base, k)])
            pltpu.sync_copy(ib_h.at[pl.ds(base, k)], idx_v)
            pltpu.async_copy(table_h.at[idx_v], rows_v, sem).wait()
            pltpu.sync_copy(rows_v, ob_h.at[pl.ds(base, k)])
            return 0

        lax.fori_loop(0, n_ch, step, 0)

    return body(table, idx_a, idx_b)


def _translate_loop(idx_v, idx2_v, k, half, core, s, dspread=984):
    def tr(i, _):
        v = idx_v[pl.ds(i * LANES, LANES)]
        li = v - core * half
        ok = (li >= 0) & (li < half)
        dummy = half + ((i * LANES + s * 64) % dspread) + lax.iota(jnp.int32, LANES)
        idx2_v[pl.ds(i * LANES, LANES)] = jnp.where(ok, li, dummy)
        return 0

    lax.fori_loop(0, k // LANES, tr, 0)


def _sc_segsum_entity(res, head, zeros64):
    e, c = res.shape
    n_ent = 50000
    half = n_ent // NC
    hpad = half + 1000
    k = 400
    per_s = e // NS
    n_ch = per_s // k
    assert per_s % k == 0
    zk = 1000

    @functools.partial(
        pl.kernel,
        mesh=plsc.VectorSubcoreMesh(**_MESH),
        compiler_params=pltpu.CompilerParams(use_tc_tiling_on_sc=False),
        out_type=jax.ShapeDtypeStruct((n_ent, c), jnp.float32),
        scratch_types=[
            pltpu.VMEM((k,), jnp.int32),
            pltpu.VMEM((k,), jnp.int32),
            pltpu.VMEM((k, c), jnp.float32),
            pltpu.VMEM_SHARED((hpad, c), jnp.float32),
        ],
    )
    def body(res_h, head_h, zeros_h, out_h, idx_v, idx2_v, rows_v, acc):
        core = lax.axis_index("c")
        s = lax.axis_index("s")

        def zstep(i, _):
            j = s + i * NS
            @pl.when(j < hpad // zk)
            def _():
                pltpu.sync_copy(zeros_h, acc.at[pl.ds(j * zk, zk)])
            return 0
        lax.fori_loop(0, (hpad // zk + NS - 1) // NS, zstep, 0)
        plsc.subcore_barrier()

        def step(j, _):
            base = s * per_s + j * k
            pltpu.sync_copy(head_h.at[pl.ds(base, k)], idx_v)
            _translate_loop(idx_v, idx2_v, k, half, core, s)
            pltpu.sync_copy(res_h.at[pl.ds(base, k)], rows_v)
            pltpu.sync_copy(rows_v, acc.at[idx2_v], add=True)
            return 0
        lax.fori_loop(0, n_ch, step, 0)
        plsc.subcore_barrier()

        def ostep(i, _):
            j = s + i * NS
            @pl.when(j < half // zk)
            def _():
                pltpu.sync_copy(acc.at[pl.ds(j * zk, zk)],
                                out_h.at[pl.ds(core * half + j * zk, zk)])
            return 0
        lax.fori_loop(0, (half // zk + NS - 1) // NS, ostep, 0)

    return body(res, head, zeros64)


def _sc_counts(head, zeros16):
    e = head.shape[0]
    n_ent = 50000
    half = n_ent // NC
    hpad = half + 1000
    k = 400
    per_s = e // NS
    n_ch = per_s // k
    zk = 1000

    @functools.partial(
        pl.kernel,
        mesh=plsc.VectorSubcoreMesh(**_MESH),
        compiler_params=pltpu.CompilerParams(use_tc_tiling_on_sc=False),
        out_type=jax.ShapeDtypeStruct((n_ent, 16), jnp.float32),
        scratch_types=[
            pltpu.VMEM((k,), jnp.int32),
            pltpu.VMEM((k,), jnp.int32),
            pltpu.VMEM((k, 16), jnp.float32),
            pltpu.VMEM_SHARED((hpad, 16), jnp.float32),
        ],
    )
    def body(head_h, zeros_h, out_h, idx_v, idx2_v, ones_v, acc):
        core = lax.axis_index("c")
        s = lax.axis_index("s")

        def fill(r, _):
            ones_v[r, pl.ds(0, 16)] = jnp.full((16,), 1.0, jnp.float32)
            return 0
        lax.fori_loop(0, k, fill, 0)

        def zstep(i, _):
            j = s + i * NS
            @pl.when(j < hpad // zk)
            def _():
                pltpu.sync_copy(zeros_h, acc.at[pl.ds(j * zk, zk)])
            return 0
        lax.fori_loop(0, (hpad // zk + NS - 1) // NS, zstep, 0)
        plsc.subcore_barrier()

        def step(j, _):
            base = s * per_s + j * k
            pltpu.sync_copy(head_h.at[pl.ds(base, k)], idx_v)
            _translate_loop(idx_v, idx2_v, k, half, core, s)
            pltpu.sync_copy(ones_v, acc.at[idx2_v], add=True)
            return 0
        lax.fori_loop(0, n_ch, step, 0)
        plsc.subcore_barrier()

        def ostep(i, _):
            j = s + i * NS
            @pl.when(j < half // zk)
            def _():
                pltpu.sync_copy(acc.at[pl.ds(j * zk, zk)],
                                out_h.at[pl.ds(core * half + j * zk, zk)])
            return 0
        lax.fori_loop(0, (half // zk + NS - 1) // NS, ostep, 0)

    return body(head, zeros16)


def _sc_item_agg(user_tab, row_g, col_s, zeros64):
    nnzp = row_g.shape[0]
    c = user_tab.shape[1]
    n_items = 20000
    ipad = n_items + 1000
    k = 520
    per_w = nnzp // (NC * NS)
    n_ch = per_w // k
    assert per_w % k == 0
    zk = 1000

    @functools.partial(
        pl.kernel,
        mesh=plsc.VectorSubcoreMesh(**_MESH),
        compiler_params=pltpu.CompilerParams(use_tc_tiling_on_sc=False),
        out_type=jax.ShapeDtypeStruct((NC, n_items, c), jnp.float32),
        scratch_types=[
            pltpu.VMEM((k,), jnp.int32),
            pltpu.VMEM((k, c), jnp.float32),
            pltpu.VMEM_SHARED((ipad, c), jnp.float32),
            pltpu.SemaphoreType.DMA,
        ],
    )
    def body(tab_h, rg_h, cs_h, zeros_h, out_h, idx_v, rows_v, acc, sem):
        core = lax.axis_index("c")
        s = lax.axis_index("s")

        def zstep(i, _):
            j = s + i * NS
            @pl.when(j < ipad // zk)
            def _():
                pltpu.sync_copy(zeros_h, acc.at[pl.ds(j * zk, zk)])
            return 0
        lax.fori_loop(0, (ipad // zk + NS - 1) // NS, zstep, 0)
        plsc.subcore_barrier()

        def step(j, _):
            base = (core * NS + s) * per_w + j * k
            pltpu.sync_copy(rg_h.at[pl.ds(base, k)], idx_v)
            pltpu.async_copy(tab_h.at[idx_v], rows_v, sem).wait()
            pltpu.sync_copy(cs_h.at[pl.ds(base, k)], idx_v)
            pltpu.sync_copy(rows_v, acc.at[idx_v], add=True)
            return 0
        lax.fori_loop(0, n_ch, step, 0)
        plsc.subcore_barrier()

        def ostep(i, _):
            j = s + i * NS
            @pl.when(j < n_items // zk)
            def _():
                pltpu.sync_copy(acc.at[pl.ds(j * zk, zk)],
                                out_h.at[core, pl.ds(j * zk, zk)])
            return 0
        lax.fori_loop(0, (n_items // zk + NS - 1) // NS, ostep, 0)

    return body(user_tab, row_g, col_s, zeros64)


def _sc_user_agg(fusion_tab, col_g, row_s, val_p, zeros64):
    nnzp = col_g.shape[0]
    c = fusion_tab.shape[1]
    n_users = 50000
    half = n_users // NC
    hpad = half + 600
    k = 400
    per_s = nnzp // NS
    n_ch = per_s // k
    assert per_s % k == 0
    zk = 1000

    @functools.partial(
        pl.kernel,
        mesh=plsc.VectorSubcoreMesh(**_MESH),
        compiler_params=pltpu.CompilerParams(use_tc_tiling_on_sc=False, needs_layout_passes=False),
        out_type=jax.ShapeDtypeStruct((n_users, c), jnp.float32),
        scratch_types=[
            pltpu.VMEM((k,), jnp.int32),
            pltpu.VMEM((k,), jnp.int32),
            pltpu.VMEM((k,), jnp.float32),
            pltpu.VMEM((k, c), jnp.float32),
            pltpu.VMEM_SHARED((hpad, c), jnp.float32),
            pltpu.SemaphoreType.DMA,
        ],
    )
    def body(tab_h, cg_h, rs_h, val_h, zeros_h, out_h,
             idx_v, idx2_v, val_v, rows_v, acc, sem):
        core = lax.axis_index("c")
        s = lax.axis_index("s")

        def zstep(i, _):
            j = s + i * NS
            @pl.when(j < hpad // zk)
            def _():
                pltpu.sync_copy(zeros_h, acc.at[pl.ds(j * zk, zk)])
            return 0
        lax.fori_loop(0, (hpad // zk + NS - 1) // NS, zstep, 0)
        plsc.subcore_barrier()

        def step(j, _):
            base = s * per_s + j * k
            pltpu.sync_copy(cg_h.at[pl.ds(base, k)], idx_v)
            pltpu.async_copy(tab_h.at[idx_v], rows_v, sem).wait()
            pltpu.sync_copy(val_h.at[pl.ds(base, k)], val_v)

            def scale(r, _):
                sv = plsc.load_gather(val_v, [jnp.zeros((16,), jnp.int32) + r])
                for q in range(4):
                    rows_v[r, pl.ds(q * 16, 16)] = rows_v[r, pl.ds(q * 16, 16)] * sv
                return 0
            lax.fori_loop(0, k, scale, 0)

            pltpu.sync_copy(rs_h.at[pl.ds(base, k)], idx_v)
            _translate_loop(idx_v, idx2_v, k, half, core, s, dspread=584)
            pltpu.sync_copy(rows_v, acc.at[idx2_v], add=True)
            return 0
        lax.fori_loop(0, n_ch, step, 0)
        plsc.subcore_barrier()

        def ostep(i, _):
            j = s + i * NS
            @pl.when(j < half // zk)
            def _():
                pltpu.sync_copy(acc.at[pl.ds(j * zk, zk)],
                                out_h.at[pl.ds(core * half + j * zk, zk)])
            return 0
        lax.fori_loop(0, (half // zk + NS - 1) // NS, ostep, 0)

    return body(fusion_tab, col_g, row_s, val_p, zeros64)


def _l2norm(x):
    n = jnp.maximum(jnp.sqrt(jnp.sum(x * x, axis=-1, keepdims=True)), 1e-12)
    return x / n


def kernel(user_emb, entity_emb, item_emb_cf, relation_weight, gate1_w, gate2_w,
           mat_val, edge_index, edge_type, mat_row, mat_col):
    n_entities = entity_emb.shape[0]
    n_users = user_emb.shape[0]
    n_items = item_emb_cf.shape[0]
    c = entity_emb.shape[1]
    nnz = mat_row.shape[0]
    head = edge_index[0]
    tail = edge_index[1]

    rw_pad = jnp.zeros((16, c), jnp.float32).at[: relation_weight.shape[0]].set(relation_weight)
    zeros64 = jnp.zeros((1000, c), jnp.float32)
    zeros16 = jnp.zeros((1000, 16), jnp.float32)

    nnzp = 416000
    npad = nnzp - nnz
    ar = jnp.arange(npad, dtype=jnp.int32)
    row_g = jnp.concatenate([mat_row, ar % n_users])
    col_s = jnp.concatenate([mat_col, n_items + ar % 1000])
    col_g = jnp.concatenate([mat_col, ar % n_items])
    row_s = jnp.concatenate([mat_row, jnp.full((npad,), n_users, jnp.int32)])
    val_p = jnp.concatenate([mat_val, jnp.zeros((npad,), jnp.float32)])

    cnt16 = _sc_counts(head, zeros16)
    inv_cnt = 1.0 / jnp.maximum(cnt16[:, 0], 1.0)

    e_res, u_res, i_res = entity_emb, user_emb, item_emb_cf
    cur_e, cur_u, cur_i = entity_emb, user_emb, item_emb_cf
    n_hops = gate1_w.shape[0]
    for hop in range(n_hops):
        head_emb, tail_emb = _sc_gather2(cur_e, head, tail)
        res = _edge_transform(head_emb, tail_emb, edge_type, rw_pad)
        sums = _sc_segsum_entity(res, head, zeros64)
        entity_agg = sums * inv_cnt[:, None]
        item_parts = _sc_item_agg(cur_u, row_g, col_s, zeros64)
        item_agg_cf = item_parts[0] + item_parts[1]
        item_emb_kg = cur_e[:n_items]
        gi = jax.nn.sigmoid(cur_i @ gate1_w[hop].T + item_emb_kg @ gate2_w[hop].T)
        item_fusion = gi * cur_i + (1.0 - gi) * item_emb_kg
        user_agg = _sc_user_agg(item_fusion, col_g, row_s, val_p, zeros64)
        cur_e = _l2norm(entity_agg)
        cur_u = _l2norm(user_agg)
        cur_i = _l2norm(item_agg_cf)
        e_res = e_res + cur_e
        u_res = u_res + cur_u
        i_res = i_res + cur_i
    return (e_res, u_res, i_res)


# speedup vs baseline: 1.7068x; 1.7068x over previous
import functools

import jax
import jax.numpy as jnp
from jax import lax
from jax.experimental import pallas as pl
from jax.experimental.pallas import tpu as pltpu
from jax.experimental.pallas import tpu_sc as plsc

EPS = 1e-10
MAX_NORM = 1.0 - 1e-5

NC = 2
NS = 16
LANES = 16

_MESH = dict(core_axis_name="c", subcore_axis_name="s", num_cores=NC,
             num_subcores=NS)


def _rownorm(x):
    return jnp.sqrt(jnp.sum(x * x, axis=-1, keepdims=True))


def _project(x):
    norm = jnp.maximum(_rownorm(x), EPS)
    scale = jnp.where(norm > MAX_NORM, MAX_NORM / norm, 1.0)
    return x * scale


def _mobius_add(x, y):
    x2 = jnp.sum(x * x, axis=-1, keepdims=True)
    y2 = jnp.sum(y * y, axis=-1, keepdims=True)
    xy = jnp.sum(x * y, axis=-1, keepdims=True)
    num = (1.0 + 2.0 * xy + y2) * x + (1.0 - x2) * y
    den = jnp.maximum(1.0 + 2.0 * xy + x2 * y2, EPS)
    return num / den


def _edge_coeffs(h2, t2, r2, d_ht, d_hr, d_tr):
    un_h = jnp.maximum(jnp.sqrt(h2), EPS)
    f = jnp.tanh(un_h) / un_h
    nh = jnp.sqrt(f * f * h2)
    f = f * jnp.where(nh > MAX_NORM, MAX_NORM / nh, 1.0)
    hh2 = f * f * h2
    two_over_lam = jnp.maximum(1.0 - hh2, EPS)
    hl = 1.0 / two_over_lam

    def sec(v2):
        un = jnp.maximum(jnp.sqrt(v2), EPS)
        return jnp.tanh(hl * un) / un

    def mobius(x2, y2, xy):
        den = jnp.maximum(1.0 + 2.0 * xy + x2 * y2, EPS)
        inv = 1.0 / den
        return (1.0 + 2.0 * xy + y2) * inv, (1.0 - x2) * inv

    def proj2(n2):
        n = jnp.sqrt(n2)
        return jnp.where(n > MAX_NORM, MAX_NORM / n, 1.0)

    g_t = sec(t2)
    st2 = g_t * g_t * t2
    A, B = mobius(hh2, st2, f * g_t * d_ht)
    a1, b1 = A * f, B * g_t
    ht2 = a1 * a1 * h2 + b1 * b1 * t2 + 2.0 * a1 * b1 * d_ht
    s = proj2(ht2)
    a1, b1, ht2 = a1 * s, b1 * s, ht2 * s * s

    g_r = sec(r2)
    sr2 = g_r * g_r * r2
    A, B = mobius(hh2, sr2, f * g_r * d_hr)
    a2, c2 = A * f, B * g_r
    hr2 = a2 * a2 * h2 + c2 * c2 * r2 + 2.0 * a2 * c2 * d_hr
    s = proj2(hr2)
    a2, c2, hr2 = a2 * s, c2 * s, hr2 * s * s

    xy = a1 * a2 * h2 + a1 * c2 * d_hr + b1 * a2 * d_ht + b1 * c2 * d_tr
    A, B = mobius(ht2, hr2, xy)
    al, be, ga = A * a1 + B * a2, A * b1, B * c2
    m2 = (al * al * h2 + be * be * t2 + ga * ga * r2
          + 2.0 * (al * be * d_ht + al * ga * d_hr + be * ga * d_tr))
    s = proj2(m2)
    al, be, ga, m2 = al * s, be * s, ga * s, m2 * s * s

    xy = -f * (al * h2 + be * d_ht + ga * d_hr)
    A, B = mobius(hh2, m2, xy)
    p, q, w = B * al - A * f, B * be, B * ga
    sub2 = (p * p * h2 + q * q * t2 + w * w * r2
            + 2.0 * (p * q * d_ht + p * w * d_hr + q * w * d_tr))

    sn = jnp.maximum(jnp.sqrt(sub2), EPS)
    snc = jnp.minimum(sn, MAX_NORM)
    artanh = 0.5 * (jnp.log1p(snc) - jnp.log1p(-snc))
    k0 = two_over_lam * artanh / sn
    return k0 * p, k0 * q, k0 * w


def _edge_math_block(head_emb, tail_emb, rel_emb):
    dot = lambda x, y: jnp.sum(x * y, axis=-1, keepdims=True)
    z = jnp.zeros_like(dot(head_emb, head_emb))
    g = jnp.concatenate(
        [dot(head_emb, head_emb), dot(tail_emb, tail_emb),
         dot(rel_emb, rel_emb), dot(head_emb, tail_emb),
         dot(head_emb, rel_emb), dot(tail_emb, rel_emb), z, z], axis=1)
    gt = g.T
    P, Q, W = _edge_coeffs(gt[0:1], gt[1:2], gt[2:3], gt[3:4], gt[4:5], gt[5:6])
    pqw = jnp.concatenate([P, Q, W, P, P, P, P, P], axis=0).T
    return (pqw[:, 0:1] * head_emb + pqw[:, 1:2] * tail_emb
            + pqw[:, 2:3] * rel_emb)


def _edge_kernel(head_ref, tail_ref, et_ref, rw_ref, out_ref):
    rel = et_ref[0, 0, :] - 1
    oh = (rel[:, None] == lax.iota(jnp.int32, 16)[None, :]).astype(jnp.float32)
    rel_emb = jnp.dot(oh, rw_ref[...], preferred_element_type=jnp.float32)
    out_ref[...] = _edge_math_block(head_ref[...], tail_ref[...], rel_emb)


def _pick_block(e, target=2000):
    best = 8
    for b in range(8, min(e, 4096) + 1, 8):
        if e % b == 0 and abs(b - target) < abs(best - target):
            best = b
    return best


def _edge_transform(head_emb, tail_emb, edge_type, rw_pad):
    e, c = head_emb.shape
    be = _pick_block(e)
    nb = e // be
    et3 = edge_type.reshape(nb, 1, be)
    return pl.pallas_call(
        _edge_kernel,
        grid=(nb,),
        in_specs=[
            pl.BlockSpec((be, c), lambda i: (i, 0)),
            pl.BlockSpec((be, c), lambda i: (i, 0)),
            pl.BlockSpec((1, 1, be), lambda i: (i, 0, 0)),
            pl.BlockSpec((16, c), lambda i: (0, 0)),
        ],
        out_specs=pl.BlockSpec((be, c), lambda i: (i, 0)),
        out_shape=jax.ShapeDtypeStruct((e, c), jnp.float32),
    )(head_emb, tail_emb, et3, rw_pad)


def _wid():
    return lax.axis_index("s") * NC + lax.axis_index("c")


def _sc_gather2(table, idx_a, idx_b):
    e = idx_a.shape[0]
    c = table.shape[1]
    k = 1000
    per_w = e // (NC * NS)
    n_ch = per_w // k
    assert per_w % k == 0

    @functools.partial(
        pl.kernel,
        mesh=plsc.VectorSubcoreMesh(**_MESH),
        compiler_params=pltpu.CompilerParams(use_tc_tiling_on_sc=False),
        out_type=[jax.ShapeDtypeStruct((e, c), jnp.float32),
                  jax.ShapeDtypeStruct((e, c), jnp.float32)],
        scratch_types=[
            pltpu.VMEM((k,), jnp.int32),
            pltpu.VMEM((k, c), jnp.float32),
            pltpu.SemaphoreType.DMA,
        ],
    )
    def body(table_h, ia_h, ib_h, oa_h, ob_h, idx_v, rows_v, sem):
        base0 = _wid() * per_w

        def step(j, _):
            base = base0 + j * k
            pltpu.sync_copy(ia_h.at[pl.ds(base, k)], idx_v)
            pltpu.async_copy(table_h.at[idx_v], rows_v, sem).wait()
            pltpu.sync_copy(rows_v, oa_h.at[pl.ds(base, k)])
            pltpu.sync_copy(ib_h.at[pl.ds(base, k)], idx_v)
            pltpu.async_copy(table_h.at[idx_v], rows_v, sem).wait()
            pltpu.sync_copy(rows_v, ob_h.at[pl.ds(base, k)])
            return 0

        lax.fori_loop(0, n_ch, step, 0)

    return body(table, idx_a, idx_b)


def _translate_loop(idx_v, idx2_v, k, half, core, s, dspread=984):
    def tr(i, _):
        v = idx_v[pl.ds(i * LANES, LANES)]
        li = v - core * half
        ok = (li >= 0) & (li < half)
        dummy = half + ((i * LANES + s * 64) % dspread) + lax.iota(jnp.int32, LANES)
        idx2_v[pl.ds(i * LANES, LANES)] = jnp.where(ok, li, dummy)
        return 0

    lax.fori_loop(0, k // LANES, tr, 0)


def _sc_segsum_entity(res, head, zeros64):
    e, c = res.shape
    n_ent = 50000
    half = n_ent // NC
    hpad = half + 1000
    k = 400
    per_s = e // NS
    n_ch = per_s // k
    assert per_s % k == 0
    zk = 1000

    @functools.partial(
        pl.kernel,
        mesh=plsc.VectorSubcoreMesh(**_MESH),
        compiler_params=pltpu.CompilerParams(use_tc_tiling_on_sc=False),
        out_type=jax.ShapeDtypeStruct((n_ent, c), jnp.float32),
        scratch_types=[
            pltpu.VMEM((k,), jnp.int32),
            pltpu.VMEM((k,), jnp.int32),
            pltpu.VMEM((k, c), jnp.float32),
            pltpu.VMEM_SHARED((hpad, c), jnp.float32),
        ],
    )
    def body(res_h, head_h, zeros_h, out_h, idx_v, idx2_v, rows_v, acc):
        core = lax.axis_index("c")
        s = lax.axis_index("s")

        def zstep(i, _):
            j = s + i * NS
            @pl.when(j < hpad // zk)
            def _():
                pltpu.sync_copy(zeros_h, acc.at[pl.ds(j * zk, zk)])
            return 0
        lax.fori_loop(0, (hpad // zk + NS - 1) // NS, zstep, 0)
        plsc.subcore_barrier()

        def step(j, _):
            base = s * per_s + j * k
            pltpu.sync_copy(head_h.at[pl.ds(base, k)], idx_v)
            _translate_loop(idx_v, idx2_v, k, half, core, s)
            pltpu.sync_copy(res_h.at[pl.ds(base, k)], rows_v)
            pltpu.sync_copy(rows_v, acc.at[idx2_v], add=True)
            return 0
        lax.fori_loop(0, n_ch, step, 0)
        plsc.subcore_barrier()

        def ostep(i, _):
            j = s + i * NS
            @pl.when(j < half // zk)
            def _():
                pltpu.sync_copy(acc.at[pl.ds(j * zk, zk)],
                                out_h.at[pl.ds(core * half + j * zk, zk)])
            return 0
        lax.fori_loop(0, (half // zk + NS - 1) // NS, ostep, 0)

    return body(res, head, zeros64)


def _sc_counts(head, zeros16):
    e = head.shape[0]
    n_ent = 50000
    half = n_ent // NC
    hpad = half + 1000
    k = 400
    per_s = e // NS
    n_ch = per_s // k
    zk = 1000

    @functools.partial(
        pl.kernel,
        mesh=plsc.VectorSubcoreMesh(**_MESH),
        compiler_params=pltpu.CompilerParams(use_tc_tiling_on_sc=False),
        out_type=jax.ShapeDtypeStruct((n_ent, 16), jnp.float32),
        scratch_types=[
            pltpu.VMEM((k,), jnp.int32),
            pltpu.VMEM((k,), jnp.int32),
            pltpu.VMEM((k, 16), jnp.float32),
            pltpu.VMEM_SHARED((hpad, 16), jnp.float32),
        ],
    )
    def body(head_h, zeros_h, out_h, idx_v, idx2_v, ones_v, acc):
        core = lax.axis_index("c")
        s = lax.axis_index("s")

        def fill(r, _):
            ones_v[r, pl.ds(0, 16)] = jnp.full((16,), 1.0, jnp.float32)
            return 0
        lax.fori_loop(0, k, fill, 0)

        def zstep(i, _):
            j = s + i * NS
            @pl.when(j < hpad // zk)
            def _():
                pltpu.sync_copy(zeros_h, acc.at[pl.ds(j * zk, zk)])
            return 0
        lax.fori_loop(0, (hpad // zk + NS - 1) // NS, zstep, 0)
        plsc.subcore_barrier()

        def step(j, _):
            base = s * per_s + j * k
            pltpu.sync_copy(head_h.at[pl.ds(base, k)], idx_v)
            _translate_loop(idx_v, idx2_v, k, half, core, s)
            pltpu.sync_copy(ones_v, acc.at[idx2_v], add=True)
            return 0
        lax.fori_loop(0, n_ch, step, 0)
        plsc.subcore_barrier()

        def ostep(i, _):
            j = s + i * NS
            @pl.when(j < half // zk)
            def _():
                pltpu.sync_copy(acc.at[pl.ds(j * zk, zk)],
                                out_h.at[pl.ds(core * half + j * zk, zk)])
            return 0
        lax.fori_loop(0, (half // zk + NS - 1) // NS, ostep, 0)

    return body(head, zeros16)


def _sc_item_agg(user_tab, row_g, col_s, zeros64):
    nnzp = row_g.shape[0]
    c = user_tab.shape[1]
    n_items = 20000
    ipad = n_items + 1000
    k = 520
    per_w = nnzp // (NC * NS)
    n_ch = per_w // k
    assert per_w % k == 0
    zk = 1000

    @functools.partial(
        pl.kernel,
        mesh=plsc.VectorSubcoreMesh(**_MESH),
        compiler_params=pltpu.CompilerParams(use_tc_tiling_on_sc=False),
        out_type=jax.ShapeDtypeStruct((NC, n_items, c), jnp.float32),
        scratch_types=[
            pltpu.VMEM((k,), jnp.int32),
            pltpu.VMEM((k, c), jnp.float32),
            pltpu.VMEM_SHARED((ipad, c), jnp.float32),
            pltpu.SemaphoreType.DMA,
        ],
    )
    def body(tab_h, rg_h, cs_h, zeros_h, out_h, idx_v, rows_v, acc, sem):
        core = lax.axis_index("c")
        s = lax.axis_index("s")

        def zstep(i, _):
            j = s + i * NS
            @pl.when(j < ipad // zk)
            def _():
                pltpu.sync_copy(zeros_h, acc.at[pl.ds(j * zk, zk)])
            return 0
        lax.fori_loop(0, (ipad // zk + NS - 1) // NS, zstep, 0)
        plsc.subcore_barrier()

        def step(j, _):
            base = (core * NS + s) * per_w + j * k
            pltpu.sync_copy(rg_h.at[pl.ds(base, k)], idx_v)
            pltpu.async_copy(tab_h.at[idx_v], rows_v, sem).wait()
            pltpu.sync_copy(cs_h.at[pl.ds(base, k)], idx_v)
            pltpu.sync_copy(rows_v, acc.at[idx_v], add=True)
            return 0
        lax.fori_loop(0, n_ch, step, 0)
        plsc.subcore_barrier()

        def ostep(i, _):
            j = s + i * NS
            @pl.when(j < n_items // zk)
            def _():
                pltpu.sync_copy(acc.at[pl.ds(j * zk, zk)],
                                out_h.at[core, pl.ds(j * zk, zk)])
            return 0
        lax.fori_loop(0, (n_items // zk + NS - 1) // NS, ostep, 0)

    return body(user_tab, row_g, col_s, zeros64)


def _sc_user_agg(fusion_tab, col_g, row_s, val_p, zeros64):
    nnzp = col_g.shape[0]
    c = fusion_tab.shape[1]
    n_users = 50000
    half = n_users // NC
    hpad = half + 600
    k = 400
    per_s = nnzp // NS
    n_ch = per_s // k
    assert per_s % k == 0
    zk = 1000

    @functools.partial(
        pl.kernel,
        mesh=plsc.VectorSubcoreMesh(**_MESH),
        compiler_params=pltpu.CompilerParams(use_tc_tiling_on_sc=False, needs_layout_passes=False),
        out_type=jax.ShapeDtypeStruct((n_users, c), jnp.float32),
        scratch_types=[
            pltpu.VMEM((k,), jnp.int32),
            pltpu.VMEM((k,), jnp.int32),
            pltpu.VMEM((k,), jnp.float32),
            pltpu.VMEM((k, c), jnp.float32),
            pltpu.VMEM_SHARED((hpad, c), jnp.float32),
            pltpu.SemaphoreType.DMA,
        ],
    )
    def body(tab_h, cg_h, rs_h, val_h, zeros_h, out_h,
             idx_v, idx2_v, val_v, rows_v, acc, sem):
        core = lax.axis_index("c")
        s = lax.axis_index("s")

        def zstep(i, _):
            j = s + i * NS
            @pl.when(j < hpad // zk)
            def _():
                pltpu.sync_copy(zeros_h, acc.at[pl.ds(j * zk, zk)])
            return 0
        lax.fori_loop(0, (hpad // zk + NS - 1) // NS, zstep, 0)
        plsc.subcore_barrier()

        def step(j, _):
            base = s * per_s + j * k
            pltpu.sync_copy(cg_h.at[pl.ds(base, k)], idx_v)
            pltpu.async_copy(tab_h.at[idx_v], rows_v, sem).wait()
            pltpu.sync_copy(val_h.at[pl.ds(base, k)], val_v)

            def scale(r, _):
                sv = plsc.load_gather(val_v, [jnp.zeros((16,), jnp.int32) + r])
                for q in range(4):
                    rows_v[r, pl.ds(q * 16, 16)] = rows_v[r, pl.ds(q * 16, 16)] * sv
                return 0
            lax.fori_loop(0, k, scale, 0)

            pltpu.sync_copy(rs_h.at[pl.ds(base, k)], idx_v)
            _translate_loop(idx_v, idx2_v, k, half, core, s, dspread=584)
            pltpu.sync_copy(rows_v, acc.at[idx2_v], add=True)
            return 0
        lax.fori_loop(0, n_ch, step, 0)
        plsc.subcore_barrier()

        def ostep(i, _):
            j = s + i * NS
            @pl.when(j < half // zk)
            def _():
                pltpu.sync_copy(acc.at[pl.ds(j * zk, zk)],
                                out_h.at[pl.ds(core * half + j * zk, zk)])
            return 0
        lax.fori_loop(0, (half // zk + NS - 1) // NS, ostep, 0)

    return body(fusion_tab, col_g, row_s, val_p, zeros64)


def _l2norm(x):
    n = jnp.maximum(jnp.sqrt(jnp.sum(x * x, axis=-1, keepdims=True)), 1e-12)
    return x / n


def kernel(user_emb, entity_emb, item_emb_cf, relation_weight, gate1_w, gate2_w,
           mat_val, edge_index, edge_type, mat_row, mat_col):
    n_entities = entity_emb.shape[0]
    n_users = user_emb.shape[0]
    n_items = item_emb_cf.shape[0]
    c = entity_emb.shape[1]
    nnz = mat_row.shape[0]
    head = edge_index[0]
    tail = edge_index[1]

    rw_pad = jnp.zeros((16, c), jnp.float32).at[: relation_weight.shape[0]].set(relation_weight)
    zeros64 = jnp.zeros((1000, c), jnp.float32)
    zeros16 = jnp.zeros((1000, 16), jnp.float32)

    nnzp = 416000
    npad = nnzp - nnz
    ar = jnp.arange(npad, dtype=jnp.int32)
    row_g = jnp.concatenate([mat_row, ar % n_users])
    col_s = jnp.concatenate([mat_col, n_items + ar % 1000])
    col_g = jnp.concatenate([mat_col, ar % n_items])
    row_s = jnp.concatenate([mat_row, jnp.full((npad,), n_users, jnp.int32)])
    val_p = jnp.concatenate([mat_val, jnp.zeros((npad,), jnp.float32)])

    cnt16 = _sc_counts(head, zeros16)
    inv_cnt = 1.0 / jnp.maximum(cnt16[:, 0], 1.0)

    e_res, u_res, i_res = entity_emb, user_emb, item_emb_cf
    cur_e, cur_u, cur_i = entity_emb, user_emb, item_emb_cf
    n_hops = gate1_w.shape[0]
    for hop in range(n_hops):
        head_emb, tail_emb = _sc_gather2(cur_e, head, tail)
        res = _edge_transform(head_emb, tail_emb, edge_type, rw_pad)
        sums = _sc_segsum_entity(res, head, zeros64)
        entity_agg = sums * inv_cnt[:, None]
        item_parts = _sc_item_agg(cur_u, row_g, col_s, zeros64)
        item_agg_cf = item_parts[0] + item_parts[1]
        item_emb_kg = cur_e[:n_items]
        gi = jax.nn.sigmoid(cur_i @ gate1_w[hop].T + item_emb_kg @ gate2_w[hop].T)
        item_fusion = gi * cur_i + (1.0 - gi) * item_emb_kg
        user_agg = _sc_user_agg(item_fusion, col_g, row_s, val_p, zeros64)
        cur_e = _l2norm(entity_agg)
        cur_u = _l2norm(user_agg)
        cur_i = _l2norm(item_agg_cf)
        e_res = e_res + cur_e
        u_res = u_res + cur_u
        i_res = i_res + cur_i
    return (e_res, u_res, i_res)


# speedup vs baseline: 1.7600x; 1.0312x over previous
import functools

import jax
import jax.numpy as jnp
from jax import lax
from jax.experimental import pallas as pl
from jax.experimental.pallas import tpu as pltpu
from jax.experimental.pallas import tpu_sc as plsc

EPS = 1e-10
MAX_NORM = 1.0 - 1e-5

NC = 2
NS = 16
LANES = 16

_MESH = dict(core_axis_name="c", subcore_axis_name="s", num_cores=NC,
             num_subcores=NS)


def _rownorm(x):
    return jnp.sqrt(jnp.sum(x * x, axis=-1, keepdims=True))


def _project(x):
    norm = jnp.maximum(_rownorm(x), EPS)
    scale = jnp.where(norm > MAX_NORM, MAX_NORM / norm, 1.0)
    return x * scale


def _mobius_add(x, y):
    x2 = jnp.sum(x * x, axis=-1, keepdims=True)
    y2 = jnp.sum(y * y, axis=-1, keepdims=True)
    xy = jnp.sum(x * y, axis=-1, keepdims=True)
    num = (1.0 + 2.0 * xy + y2) * x + (1.0 - x2) * y
    den = jnp.maximum(1.0 + 2.0 * xy + x2 * y2, EPS)
    return num / den


def _edge_coeffs(h2, t2, r2, d_ht, d_hr, d_tr):
    un_h = jnp.maximum(jnp.sqrt(h2), EPS)
    f = jnp.tanh(un_h) / un_h
    nh = jnp.sqrt(f * f * h2)
    f = f * jnp.where(nh > MAX_NORM, MAX_NORM / nh, 1.0)
    hh2 = f * f * h2
    two_over_lam = jnp.maximum(1.0 - hh2, EPS)
    hl = 1.0 / two_over_lam

    def sec(v2):
        un = jnp.maximum(jnp.sqrt(v2), EPS)
        return jnp.tanh(hl * un) / un

    def mobius(x2, y2, xy):
        den = jnp.maximum(1.0 + 2.0 * xy + x2 * y2, EPS)
        inv = 1.0 / den
        return (1.0 + 2.0 * xy + y2) * inv, (1.0 - x2) * inv

    def proj2(n2):
        n = jnp.sqrt(n2)
        return jnp.where(n > MAX_NORM, MAX_NORM / n, 1.0)

    g_t = sec(t2)
    st2 = g_t * g_t * t2
    A, B = mobius(hh2, st2, f * g_t * d_ht)
    a1, b1 = A * f, B * g_t
    ht2 = a1 * a1 * h2 + b1 * b1 * t2 + 2.0 * a1 * b1 * d_ht
    s = proj2(ht2)
    a1, b1, ht2 = a1 * s, b1 * s, ht2 * s * s

    g_r = sec(r2)
    sr2 = g_r * g_r * r2
    A, B = mobius(hh2, sr2, f * g_r * d_hr)
    a2, c2 = A * f, B * g_r
    hr2 = a2 * a2 * h2 + c2 * c2 * r2 + 2.0 * a2 * c2 * d_hr
    s = proj2(hr2)
    a2, c2, hr2 = a2 * s, c2 * s, hr2 * s * s

    xy = a1 * a2 * h2 + a1 * c2 * d_hr + b1 * a2 * d_ht + b1 * c2 * d_tr
    A, B = mobius(ht2, hr2, xy)
    al, be, ga = A * a1 + B * a2, A * b1, B * c2
    m2 = (al * al * h2 + be * be * t2 + ga * ga * r2
          + 2.0 * (al * be * d_ht + al * ga * d_hr + be * ga * d_tr))
    s = proj2(m2)
    al, be, ga, m2 = al * s, be * s, ga * s, m2 * s * s

    xy = -f * (al * h2 + be * d_ht + ga * d_hr)
    A, B = mobius(hh2, m2, xy)
    p, q, w = B * al - A * f, B * be, B * ga
    sub2 = (p * p * h2 + q * q * t2 + w * w * r2
            + 2.0 * (p * q * d_ht + p * w * d_hr + q * w * d_tr))

    sn = jnp.maximum(jnp.sqrt(sub2), EPS)
    snc = jnp.minimum(sn, MAX_NORM)
    artanh = 0.5 * (jnp.log1p(snc) - jnp.log1p(-snc))
    k0 = two_over_lam * artanh / sn
    return k0 * p, k0 * q, k0 * w


def _edge_math_block(head_emb, tail_emb, rel_emb):
    dot = lambda x, y: jnp.sum(x * y, axis=-1, keepdims=True)
    z = jnp.zeros_like(dot(head_emb, head_emb))
    g = jnp.concatenate(
        [dot(head_emb, head_emb), dot(tail_emb, tail_emb),
         dot(rel_emb, rel_emb), dot(head_emb, tail_emb),
         dot(head_emb, rel_emb), dot(tail_emb, rel_emb), z, z], axis=1)
    gt = g.T
    P, Q, W = _edge_coeffs(gt[0:1], gt[1:2], gt[2:3], gt[3:4], gt[4:5], gt[5:6])
    pqw = jnp.concatenate([P, Q, W, P, P, P, P, P], axis=0).T
    return (pqw[:, 0:1] * head_emb + pqw[:, 1:2] * tail_emb
            + pqw[:, 2:3] * rel_emb)


def _edge_kernel(head_ref, tail_ref, et_ref, rw_ref, out_ref):
    rel = et_ref[0, 0, :] - 1
    oh = (rel[:, None] == lax.iota(jnp.int32, 16)[None, :]).astype(jnp.float32)
    rel_emb = jnp.dot(oh, rw_ref[...], preferred_element_type=jnp.float32)
    out_ref[...] = _edge_math_block(head_ref[...], tail_ref[...], rel_emb)


def _pick_block(e, target=2000):
    best = 8
    for b in range(8, min(e, 4096) + 1, 8):
        if e % b == 0 and abs(b - target) < abs(best - target):
            best = b
    return best


def _edge_transform(head_emb, tail_emb, edge_type, rw_pad):
    e, c = head_emb.shape
    be = _pick_block(e)
    nb = e // be
    et3 = edge_type.reshape(nb, 1, be)
    return pl.pallas_call(
        _edge_kernel,
        grid=(nb,),
        in_specs=[
            pl.BlockSpec((be, c), lambda i: (i, 0)),
            pl.BlockSpec((be, c), lambda i: (i, 0)),
            pl.BlockSpec((1, 1, be), lambda i: (i, 0, 0)),
            pl.BlockSpec((16, c), lambda i: (0, 0)),
        ],
        out_specs=pl.BlockSpec((be, c), lambda i: (i, 0)),
        out_shape=jax.ShapeDtypeStruct((e, c), jnp.float32),
    )(head_emb, tail_emb, et3, rw_pad)


def _wid():
    return lax.axis_index("s") * NC + lax.axis_index("c")


def _sc_gather2(table, idx_a, idx_b):
    e = idx_a.shape[0]
    c = table.shape[1]
    k = 1000
    per_w = e // (NC * NS)
    n_ch = per_w // k
    assert per_w % k == 0 and n_ch % 2 == 1

    @functools.partial(
        pl.kernel,
        mesh=plsc.VectorSubcoreMesh(**_MESH),
        compiler_params=pltpu.CompilerParams(use_tc_tiling_on_sc=False),
        out_type=[jax.ShapeDtypeStruct((e, c), jnp.float32),
                  jax.ShapeDtypeStruct((e, c), jnp.float32)],
        scratch_types=[
            pltpu.VMEM((k,), jnp.int32),
            pltpu.VMEM((k,), jnp.int32),
            pltpu.VMEM((k, c), jnp.float32),
            pltpu.VMEM((k, c), jnp.float32),
            pltpu.SemaphoreType.DMA,
            pltpu.SemaphoreType.DMA,
            pltpu.SemaphoreType.DMA,
            pltpu.SemaphoreType.DMA,
        ],
    )
    def body(table_h, ia_h, ib_h, oa_h, ob_h,
             idx0, idx1, rows0, rows1, gs0, gs1, os0, os1):
        base0 = _wid() * per_w

        def one_pass(src_h, out_h):
            def cb(j):
                return base0 + j * k

            pltpu.sync_copy(src_h.at[pl.ds(cb(0), k)], idx0)
            pltpu.async_copy(table_h.at[idx0], rows0, gs0)
            pltpu.sync_copy(src_h.at[pl.ds(cb(1), k)], idx1)
            pltpu.async_copy(table_h.at[idx1], rows1, gs1)

            def grp(g, _):
                j = 2 * g
                pltpu.make_async_copy(table_h.at[idx0], rows0, gs0).wait()
                pltpu.async_copy(rows0, out_h.at[pl.ds(cb(j), k)], os0)
                pltpu.make_async_copy(table_h.at[idx1], rows1, gs1).wait()
                pltpu.async_copy(rows1, out_h.at[pl.ds(cb(j + 1), k)], os1)
                pltpu.sync_copy(src_h.at[pl.ds(cb(j + 2), k)], idx0)
                pltpu.make_async_copy(rows0, out_h.at[pl.ds(cb(j), k)], os0).wait()
                pltpu.async_copy(table_h.at[idx0], rows0, gs0)
                pltpu.sync_copy(src_h.at[pl.ds(cb(j + 3), k)], idx1)
                pltpu.make_async_copy(rows1, out_h.at[pl.ds(cb(j + 1), k)], os1).wait()
                pltpu.async_copy(table_h.at[idx1], rows1, gs1)
                return 0

            n_grp = (n_ch - 1) // 2
            lax.fori_loop(0, n_grp - 1, grp, 0)
            jl = 2 * n_grp - 2
            pltpu.make_async_copy(table_h.at[idx0], rows0, gs0).wait()
            pltpu.async_copy(rows0, out_h.at[pl.ds(cb(jl), k)], os0)
            pltpu.make_async_copy(table_h.at[idx1], rows1, gs1).wait()
            pltpu.async_copy(rows1, out_h.at[pl.ds(cb(jl + 1), k)], os1)
            pltpu.sync_copy(src_h.at[pl.ds(cb(n_ch - 1), k)], idx0)
            pltpu.make_async_copy(rows0, out_h.at[pl.ds(cb(jl), k)], os0).wait()
            pltpu.async_copy(table_h.at[idx0], rows0, gs0)
            pltpu.make_async_copy(table_h.at[idx0], rows0, gs0).wait()
            pltpu.async_copy(rows0, out_h.at[pl.ds(cb(n_ch - 1), k)], os0)
            pltpu.make_async_copy(rows0, out_h.at[pl.ds(cb(n_ch - 1), k)], os0).wait()
            pltpu.make_async_copy(rows1, out_h.at[pl.ds(cb(jl + 1), k)], os1).wait()

        one_pass(ia_h, oa_h)
        one_pass(ib_h, ob_h)

    return body(table, idx_a, idx_b)


def _translate_loop(idx_v, idx2_v, k, half, core, s, dspread=984):
    def tr(i, _):
        v = idx_v[pl.ds(i * LANES, LANES)]
        li = v - core * half
        ok = (li >= 0) & (li < half)
        dummy = half + ((i * LANES + s * 64) % dspread) + lax.iota(jnp.int32, LANES)
        idx2_v[pl.ds(i * LANES, LANES)] = jnp.where(ok, li, dummy)
        return 0

    lax.fori_loop(0, k // LANES, tr, 0)


def _sc_segsum_entity(res, head, zeros64):
    e, c = res.shape
    ch = c // 2
    n_ent = 50000
    half = n_ent // NC
    dsp = 64
    hpad = half + dsp
    k = 400
    per_s = e // NS
    n_ch = per_s // k
    assert per_s % k == 0
    zk = 1000

    @functools.partial(
        pl.kernel,
        mesh=plsc.VectorSubcoreMesh(**_MESH),
        compiler_params=pltpu.CompilerParams(use_tc_tiling_on_sc=False),
        out_type=jax.ShapeDtypeStruct((n_ent, c), jnp.float32),
        scratch_types=[
            pltpu.VMEM((k,), jnp.int32),
            pltpu.VMEM((k,), jnp.int32),
            pltpu.VMEM((k,), jnp.int32),
            pltpu.VMEM((k,), jnp.int32),
            pltpu.VMEM((k, ch), jnp.float32),
            pltpu.VMEM((k, ch), jnp.float32),
            pltpu.VMEM_SHARED((hpad, ch), jnp.float32),
            pltpu.SemaphoreType.DMA,
            pltpu.SemaphoreType.DMA,
            pltpu.SemaphoreType.DMA,
            pltpu.SemaphoreType.DMA,
        ],
    )
    def body(res_h, head_h, zeros_h, out_h,
             idx0, idx1, ix0, ix1, rows0, rows1, acc, rs0, rs1, ss0, ss1):
        core = lax.axis_index("c")
        s = lax.axis_index("s")

        def col_pass(coff):
            def zstep(i, _):
                j = s + i * NS
                @pl.when(j < half // zk)
                def _():
                    pltpu.sync_copy(zeros_h.at[:, pl.ds(0, ch)],
                                    acc.at[pl.ds(j * zk, zk)])
                return 0
            lax.fori_loop(0, (half // zk + NS - 1) // NS, zstep, 0)
            plsc.subcore_barrier()

            def cb(j):
                return s * per_s + j * k

            def load(j, idx, ix, rows, rsem):
                pltpu.sync_copy(head_h.at[pl.ds(cb(j), k)], idx)
                _translate_loop(idx, ix, k, half, core, s, dspread=dsp - 16)
                pltpu.async_copy(res_h.at[pl.ds(cb(j), k), pl.ds(coff, ch)],
                                 rows, rsem)

            load(0, idx0, ix0, rows0, rs0)
            load(1, idx1, ix1, rows1, rs1)

            def grp(g, _):
                j = 2 * g
                pltpu.make_async_copy(
                    res_h.at[pl.ds(cb(j), k), pl.ds(coff, ch)], rows0, rs0).wait()
                pltpu.async_copy(rows0, acc.at[ix0], ss0, add=True)
                pltpu.make_async_copy(
                    res_h.at[pl.ds(cb(j + 1), k), pl.ds(coff, ch)], rows1, rs1).wait()
                pltpu.async_copy(rows1, acc.at[ix1], ss1, add=True)
                pltpu.sync_copy(head_h.at[pl.ds(cb(j + 2), k)], idx0)
                pltpu.make_async_copy(rows0, acc.at[ix0], ss0).wait()
                _translate_loop(idx0, ix0, k, half, core, s, dspread=dsp - 16)
                pltpu.async_copy(res_h.at[pl.ds(cb(j + 2), k), pl.ds(coff, ch)],
                                 rows0, rs0)
                pltpu.sync_copy(head_h.at[pl.ds(cb(j + 3), k)], idx1)
                pltpu.make_async_copy(rows1, acc.at[ix1], ss1).wait()
                _translate_loop(idx1, ix1, k, half, core, s, dspread=dsp - 16)
                pltpu.async_copy(res_h.at[pl.ds(cb(j + 3), k), pl.ds(coff, ch)],
                                 rows1, rs1)
                return 0

            n_grp = (n_ch - 1) // 2
            lax.fori_loop(0, n_grp - 1, grp, 0)
            jl = 2 * n_grp - 2
            pltpu.make_async_copy(
                res_h.at[pl.ds(cb(jl), k), pl.ds(coff, ch)], rows0, rs0).wait()
            pltpu.async_copy(rows0, acc.at[ix0], ss0, add=True)
            pltpu.make_async_copy(
                res_h.at[pl.ds(cb(jl + 1), k), pl.ds(coff, ch)], rows1, rs1).wait()
            pltpu.async_copy(rows1, acc.at[ix1], ss1, add=True)
            pltpu.sync_copy(head_h.at[pl.ds(cb(n_ch - 1), k)], idx0)
            pltpu.make_async_copy(rows0, acc.at[ix0], ss0).wait()
            _translate_loop(idx0, ix0, k, half, core, s, dspread=dsp - 16)
            pltpu.async_copy(res_h.at[pl.ds(cb(n_ch - 1), k), pl.ds(coff, ch)],
                             rows0, rs0)
            pltpu.make_async_copy(
                res_h.at[pl.ds(cb(n_ch - 1), k), pl.ds(coff, ch)], rows0, rs0).wait()
            pltpu.async_copy(rows0, acc.at[ix0], ss0, add=True)
            pltpu.make_async_copy(rows0, acc.at[ix0], ss0).wait()
            pltpu.make_async_copy(rows1, acc.at[ix1], ss1).wait()
            plsc.subcore_barrier()

            def ostep(i, _):
                j = s + i * NS
                @pl.when(j < half // zk)
                def _():
                    pltpu.sync_copy(
                        acc.at[pl.ds(j * zk, zk)],
                        out_h.at[pl.ds(core * half + j * zk, zk), pl.ds(coff, ch)])
                return 0
            lax.fori_loop(0, (half // zk + NS - 1) // NS, ostep, 0)
            plsc.subcore_barrier()

        col_pass(0)
        col_pass(ch)

    return body(res, head, zeros64)


def _sc_counts(head, zeros16):
    e = head.shape[0]
    n_ent = 50000
    half = n_ent // NC
    hpad = half + 1000
    k = 400
    per_s = e // NS
    n_ch = per_s // k
    zk = 1000

    @functools.partial(
        pl.kernel,
        mesh=plsc.VectorSubcoreMesh(**_MESH),
        compiler_params=pltpu.CompilerParams(use_tc_tiling_on_sc=False),
        out_type=jax.ShapeDtypeStruct((n_ent, 16), jnp.float32),
        scratch_types=[
            pltpu.VMEM((k,), jnp.int32),
            pltpu.VMEM((k,), jnp.int32),
            pltpu.VMEM((k, 16), jnp.float32),
            pltpu.VMEM_SHARED((hpad, 16), jnp.float32),
        ],
    )
    def body(head_h, zeros_h, out_h, idx_v, idx2_v, ones_v, acc):
        core = lax.axis_index("c")
        s = lax.axis_index("s")

        def fill(r, _):
            ones_v[r, pl.ds(0, 16)] = jnp.full((16,), 1.0, jnp.float32)
            return 0
        lax.fori_loop(0, k, fill, 0)

        def zstep(i, _):
            j = s + i * NS
            @pl.when(j < hpad // zk)
            def _():
                pltpu.sync_copy(zeros_h, acc.at[pl.ds(j * zk, zk)])
            return 0
        lax.fori_loop(0, (hpad // zk + NS - 1) // NS, zstep, 0)
        plsc.subcore_barrier()

        def step(j, _):
            base = s * per_s + j * k
            pltpu.sync_copy(head_h.at[pl.ds(base, k)], idx_v)
            _translate_loop(idx_v, idx2_v, k, half, core, s)
            pltpu.sync_copy(ones_v, acc.at[idx2_v], add=True)
            return 0
        lax.fori_loop(0, n_ch, step, 0)
        plsc.subcore_barrier()

        def ostep(i, _):
            j = s + i * NS
            @pl.when(j < half // zk)
            def _():
                pltpu.sync_copy(acc.at[pl.ds(j * zk, zk)],
                                out_h.at[pl.ds(core * half + j * zk, zk)])
            return 0
        lax.fori_loop(0, (half // zk + NS - 1) // NS, ostep, 0)

    return body(head, zeros16)


def _sc_item_agg(user_tab, row_g, col_s, zeros64):
    nnzp = row_g.shape[0]
    c = user_tab.shape[1]
    n_items = 20000
    ipad = n_items + 1000
    k = 520
    per_w = nnzp // (NC * NS)
    n_ch = per_w // k
    assert per_w % k == 0
    zk = 1000

    @functools.partial(
        pl.kernel,
        mesh=plsc.VectorSubcoreMesh(**_MESH),
        compiler_params=pltpu.CompilerParams(use_tc_tiling_on_sc=False),
        out_type=jax.ShapeDtypeStruct((NC, n_items, c), jnp.float32),
        scratch_types=[
            pltpu.VMEM((k,), jnp.int32),
            pltpu.VMEM((k, c), jnp.float32),
            pltpu.VMEM_SHARED((ipad, c), jnp.float32),
            pltpu.SemaphoreType.DMA,
        ],
    )
    def body(tab_h, rg_h, cs_h, zeros_h, out_h, idx_v, rows_v, acc, sem):
        core = lax.axis_index("c")
        s = lax.axis_index("s")

        def zstep(i, _):
            j = s + i * NS
            @pl.when(j < ipad // zk)
            def _():
                pltpu.sync_copy(zeros_h, acc.at[pl.ds(j * zk, zk)])
            return 0
        lax.fori_loop(0, (ipad // zk + NS - 1) // NS, zstep, 0)
        plsc.subcore_barrier()

        def step(j, _):
            base = (core * NS + s) * per_w + j * k
            pltpu.sync_copy(rg_h.at[pl.ds(base, k)], idx_v)
            pltpu.async_copy(tab_h.at[idx_v], rows_v, sem).wait()
            pltpu.sync_copy(cs_h.at[pl.ds(base, k)], idx_v)
            pltpu.sync_copy(rows_v, acc.at[idx_v], add=True)
            return 0
        lax.fori_loop(0, n_ch, step, 0)
        plsc.subcore_barrier()

        def ostep(i, _):
            j = s + i * NS
            @pl.when(j < n_items // zk)
            def _():
                pltpu.sync_copy(acc.at[pl.ds(j * zk, zk)],
                                out_h.at[core, pl.ds(j * zk, zk)])
            return 0
        lax.fori_loop(0, (n_items // zk + NS - 1) // NS, ostep, 0)

    return body(user_tab, row_g, col_s, zeros64)


def _sc_user_agg(fusion_tab, col_g, row_s, val_p, zeros64):
    nnzp = col_g.shape[0]
    c = fusion_tab.shape[1]
    n_users = 50000
    half = n_users // NC
    hpad = half + 600
    k = 400
    per_s = nnzp // NS
    n_ch = per_s // k
    assert per_s % k == 0
    zk = 1000

    @functools.partial(
        pl.kernel,
        mesh=plsc.VectorSubcoreMesh(**_MESH),
        compiler_params=pltpu.CompilerParams(use_tc_tiling_on_sc=False, needs_layout_passes=False),
        out_type=jax.ShapeDtypeStruct((n_users, c), jnp.float32),
        scratch_types=[
            pltpu.VMEM((k,), jnp.int32),
            pltpu.VMEM((k,), jnp.int32),
            pltpu.VMEM((k,), jnp.float32),
            pltpu.VMEM((k, c), jnp.float32),
            pltpu.VMEM_SHARED((hpad, c), jnp.float32),
            pltpu.SemaphoreType.DMA,
        ],
    )
    def body(tab_h, cg_h, rs_h, val_h, zeros_h, out_h,
             idx_v, idx2_v, val_v, rows_v, acc, sem):
        core = lax.axis_index("c")
        s = lax.axis_index("s")

        def zstep(i, _):
            j = s + i * NS
            @pl.when(j < hpad // zk)
            def _():
                pltpu.sync_copy(zeros_h, acc.at[pl.ds(j * zk, zk)])
            return 0
        lax.fori_loop(0, (hpad // zk + NS - 1) // NS, zstep, 0)
        plsc.subcore_barrier()

        def step(j, _):
            base = s * per_s + j * k
            pltpu.sync_copy(cg_h.at[pl.ds(base, k)], idx_v)
            pltpu.async_copy(tab_h.at[idx_v], rows_v, sem).wait()
            pltpu.sync_copy(val_h.at[pl.ds(base, k)], val_v)

            def scale(r, _):
                sv = plsc.load_gather(val_v, [jnp.zeros((16,), jnp.int32) + r])
                for q in range(4):
                    rows_v[r, pl.ds(q * 16, 16)] = rows_v[r, pl.ds(q * 16, 16)] * sv
                return 0
            lax.fori_loop(0, k, scale, 0)

            pltpu.sync_copy(rs_h.at[pl.ds(base, k)], idx_v)
            _translate_loop(idx_v, idx2_v, k, half, core, s, dspread=584)
            pltpu.sync_copy(rows_v, acc.at[idx2_v], add=True)
            return 0
        lax.fori_loop(0, n_ch, step, 0)
        plsc.subcore_barrier()

        def ostep(i, _):
            j = s + i * NS
            @pl.when(j < half // zk)
            def _():
                pltpu.sync_copy(acc.at[pl.ds(j * zk, zk)],
                                out_h.at[pl.ds(core * half + j * zk, zk)])
            return 0
        lax.fori_loop(0, (half // zk + NS - 1) // NS, ostep, 0)

    return body(fusion_tab, col_g, row_s, val_p, zeros64)


def _l2norm(x):
    n = jnp.maximum(jnp.sqrt(jnp.sum(x * x, axis=-1, keepdims=True)), 1e-12)
    return x / n


def kernel(user_emb, entity_emb, item_emb_cf, relation_weight, gate1_w, gate2_w,
           mat_val, edge_index, edge_type, mat_row, mat_col):
    n_entities = entity_emb.shape[0]
    n_users = user_emb.shape[0]
    n_items = item_emb_cf.shape[0]
    c = entity_emb.shape[1]
    nnz = mat_row.shape[0]
    head = edge_index[0]
    tail = edge_index[1]

    rw_pad = jnp.zeros((16, c), jnp.float32).at[: relation_weight.shape[0]].set(relation_weight)
    zeros64 = jnp.zeros((1000, c), jnp.float32)
    zeros16 = jnp.zeros((1000, 16), jnp.float32)

    nnzp = 416000
    npad = nnzp - nnz
    ar = jnp.arange(npad, dtype=jnp.int32)
    row_g = jnp.concatenate([mat_row, ar % n_users])
    col_s = jnp.concatenate([mat_col, n_items + ar % 1000])
    col_g = jnp.concatenate([mat_col, ar % n_items])
    row_s = jnp.concatenate([mat_row, jnp.full((npad,), n_users, jnp.int32)])
    val_p = jnp.concatenate([mat_val, jnp.zeros((npad,), jnp.float32)])

    cnt16 = _sc_counts(head, zeros16)
    inv_cnt = 1.0 / jnp.maximum(cnt16[:, 0], 1.0)

    e_res, u_res, i_res = entity_emb, user_emb, item_emb_cf
    cur_e, cur_u, cur_i = entity_emb, user_emb, item_emb_cf
    n_hops = gate1_w.shape[0]
    for hop in range(n_hops):
        head_emb, tail_emb = _sc_gather2(cur_e, head, tail)
        res = _edge_transform(head_emb, tail_emb, edge_type, rw_pad)
        sums = _sc_segsum_entity(res, head, zeros64)
        entity_agg = sums * inv_cnt[:, None]
        item_parts = _sc_item_agg(cur_u, row_g, col_s, zeros64)
        item_agg_cf = item_parts[0] + item_parts[1]
        item_emb_kg = cur_e[:n_items]
        gi = jax.nn.sigmoid(cur_i @ gate1_w[hop].T + item_emb_kg @ gate2_w[hop].T)
        item_fusion = gi * cur_i + (1.0 - gi) * item_emb_kg
        user_agg = _sc_user_agg(item_fusion, col_g, row_s, val_p, zeros64)
        cur_e = _l2norm(entity_agg)
        cur_u = _l2norm(user_agg)
        cur_i = _l2norm(item_agg_cf)
        e_res = e_res + cur_e
        u_res = u_res + cur_u
        i_res = i_res + cur_i
    return (e_res, u_res, i_res)


# speedup vs baseline: 1.7605x; 1.0003x over previous
import functools

import jax
import jax.numpy as jnp
from jax import lax
from jax.experimental import pallas as pl
from jax.experimental.pallas import tpu as pltpu
from jax.experimental.pallas import tpu_sc as plsc

EPS = 1e-10
MAX_NORM = 1.0 - 1e-5

NC = 2
NS = 16
LANES = 16

_MESH = dict(core_axis_name="c", subcore_axis_name="s", num_cores=NC,
             num_subcores=NS)


def _rownorm(x):
    return jnp.sqrt(jnp.sum(x * x, axis=-1, keepdims=True))


def _project(x):
    norm = jnp.maximum(_rownorm(x), EPS)
    scale = jnp.where(norm > MAX_NORM, MAX_NORM / norm, 1.0)
    return x * scale


def _mobius_add(x, y):
    x2 = jnp.sum(x * x, axis=-1, keepdims=True)
    y2 = jnp.sum(y * y, axis=-1, keepdims=True)
    xy = jnp.sum(x * y, axis=-1, keepdims=True)
    num = (1.0 + 2.0 * xy + y2) * x + (1.0 - x2) * y
    den = jnp.maximum(1.0 + 2.0 * xy + x2 * y2, EPS)
    return num / den


def _edge_coeffs(h2, t2, r2, d_ht, d_hr, d_tr):
    un_h = jnp.maximum(jnp.sqrt(h2), EPS)
    f = jnp.tanh(un_h) / un_h
    nh = jnp.sqrt(f * f * h2)
    f = f * jnp.where(nh > MAX_NORM, MAX_NORM / nh, 1.0)
    hh2 = f * f * h2
    two_over_lam = jnp.maximum(1.0 - hh2, EPS)
    hl = 1.0 / two_over_lam

    def sec(v2):
        un = jnp.maximum(jnp.sqrt(v2), EPS)
        return jnp.tanh(hl * un) / un

    def mobius(x2, y2, xy):
        den = jnp.maximum(1.0 + 2.0 * xy + x2 * y2, EPS)
        inv = 1.0 / den
        return (1.0 + 2.0 * xy + y2) * inv, (1.0 - x2) * inv

    def proj2(n2):
        n = jnp.sqrt(n2)
        return jnp.where(n > MAX_NORM, MAX_NORM / n, 1.0)

    g_t = sec(t2)
    st2 = g_t * g_t * t2
    A, B = mobius(hh2, st2, f * g_t * d_ht)
    a1, b1 = A * f, B * g_t
    ht2 = a1 * a1 * h2 + b1 * b1 * t2 + 2.0 * a1 * b1 * d_ht
    s = proj2(ht2)
    a1, b1, ht2 = a1 * s, b1 * s, ht2 * s * s

    g_r = sec(r2)
    sr2 = g_r * g_r * r2
    A, B = mobius(hh2, sr2, f * g_r * d_hr)
    a2, c2 = A * f, B * g_r
    hr2 = a2 * a2 * h2 + c2 * c2 * r2 + 2.0 * a2 * c2 * d_hr
    s = proj2(hr2)
    a2, c2, hr2 = a2 * s, c2 * s, hr2 * s * s

    xy = a1 * a2 * h2 + a1 * c2 * d_hr + b1 * a2 * d_ht + b1 * c2 * d_tr
    A, B = mobius(ht2, hr2, xy)
    al, be, ga = A * a1 + B * a2, A * b1, B * c2
    m2 = (al * al * h2 + be * be * t2 + ga * ga * r2
          + 2.0 * (al * be * d_ht + al * ga * d_hr + be * ga * d_tr))
    s = proj2(m2)
    al, be, ga, m2 = al * s, be * s, ga * s, m2 * s * s

    xy = -f * (al * h2 + be * d_ht + ga * d_hr)
    A, B = mobius(hh2, m2, xy)
    p, q, w = B * al - A * f, B * be, B * ga
    sub2 = (p * p * h2 + q * q * t2 + w * w * r2
            + 2.0 * (p * q * d_ht + p * w * d_hr + q * w * d_tr))

    sn = jnp.maximum(jnp.sqrt(sub2), EPS)
    snc = jnp.minimum(sn, MAX_NORM)
    artanh = 0.5 * (jnp.log1p(snc) - jnp.log1p(-snc))
    k0 = two_over_lam * artanh / sn
    return k0 * p, k0 * q, k0 * w


def _edge_math_block(head_emb, tail_emb, rel_emb):
    dot = lambda x, y: jnp.sum(x * y, axis=-1, keepdims=True)
    z = jnp.zeros_like(dot(head_emb, head_emb))
    g = jnp.concatenate(
        [dot(head_emb, head_emb), dot(tail_emb, tail_emb),
         dot(rel_emb, rel_emb), dot(head_emb, tail_emb),
         dot(head_emb, rel_emb), dot(tail_emb, rel_emb), z, z], axis=1)
    gt = g.T
    P, Q, W = _edge_coeffs(gt[0:1], gt[1:2], gt[2:3], gt[3:4], gt[4:5], gt[5:6])
    pqw = jnp.concatenate([P, Q, W, P, P, P, P, P], axis=0).T
    return (pqw[:, 0:1] * head_emb + pqw[:, 1:2] * tail_emb
            + pqw[:, 2:3] * rel_emb)


def _edge_kernel(head_ref, tail_ref, et_ref, rw_ref, out_ref):
    rel = et_ref[0, 0, :] - 1
    oh = (rel[:, None] == lax.iota(jnp.int32, 16)[None, :]).astype(jnp.float32)
    rel_emb = jnp.dot(oh, rw_ref[...], preferred_element_type=jnp.float32)
    out_ref[...] = _edge_math_block(head_ref[...], tail_ref[...], rel_emb)


def _pick_block(e, target=2000):
    best = 8
    for b in range(8, min(e, 4096) + 1, 8):
        if e % b == 0 and abs(b - target) < abs(best - target):
            best = b
    return best


def _edge_transform(head_emb, tail_emb, edge_type, rw_pad):
    e, c = head_emb.shape
    be = _pick_block(e)
    nb = e // be
    et3 = edge_type.reshape(nb, 1, be)
    return pl.pallas_call(
        _edge_kernel,
        grid=(nb,),
        in_specs=[
            pl.BlockSpec((be, c), lambda i: (i, 0)),
            pl.BlockSpec((be, c), lambda i: (i, 0)),
            pl.BlockSpec((1, 1, be), lambda i: (i, 0, 0)),
            pl.BlockSpec((16, c), lambda i: (0, 0)),
        ],
        out_specs=pl.BlockSpec((be, c), lambda i: (i, 0)),
        out_shape=jax.ShapeDtypeStruct((e, c), jnp.float32),
    )(head_emb, tail_emb, et3, rw_pad)


def _wid():
    return lax.axis_index("s") * NC + lax.axis_index("c")


def _sc_gather2(table, idx_a, idx_b):
    e = idx_a.shape[0]
    c = table.shape[1]
    k = 1000
    per_w = e // (NC * NS)
    n_ch = per_w // k
    assert per_w % k == 0 and n_ch % 2 == 1

    @functools.partial(
        pl.kernel,
        mesh=plsc.VectorSubcoreMesh(**_MESH),
        compiler_params=pltpu.CompilerParams(use_tc_tiling_on_sc=False),
        out_type=[jax.ShapeDtypeStruct((e, c), jnp.float32),
                  jax.ShapeDtypeStruct((e, c), jnp.float32)],
        scratch_types=[
            pltpu.VMEM((k,), jnp.int32),
            pltpu.VMEM((k,), jnp.int32),
            pltpu.VMEM((k, c), jnp.float32),
            pltpu.VMEM((k, c), jnp.float32),
            pltpu.SemaphoreType.DMA,
            pltpu.SemaphoreType.DMA,
            pltpu.SemaphoreType.DMA,
            pltpu.SemaphoreType.DMA,
        ],
    )
    def body(table_h, ia_h, ib_h, oa_h, ob_h,
             idx0, idx1, rows0, rows1, gs0, gs1, os0, os1):
        base0 = _wid() * per_w

        def one_pass(src_h, out_h):
            def cb(j):
                return base0 + j * k

            pltpu.sync_copy(src_h.at[pl.ds(cb(0), k)], idx0)
            pltpu.async_copy(table_h.at[idx0], rows0, gs0)
            pltpu.sync_copy(src_h.at[pl.ds(cb(1), k)], idx1)
            pltpu.async_copy(table_h.at[idx1], rows1, gs1)

            def grp(g, _):
                j = 2 * g
                pltpu.make_async_copy(table_h.at[idx0], rows0, gs0).wait()
                pltpu.async_copy(rows0, out_h.at[pl.ds(cb(j), k)], os0)
                pltpu.make_async_copy(table_h.at[idx1], rows1, gs1).wait()
                pltpu.async_copy(rows1, out_h.at[pl.ds(cb(j + 1), k)], os1)
                pltpu.sync_copy(src_h.at[pl.ds(cb(j + 2), k)], idx0)
                pltpu.make_async_copy(rows0, out_h.at[pl.ds(cb(j), k)], os0).wait()
                pltpu.async_copy(table_h.at[idx0], rows0, gs0)
                pltpu.sync_copy(src_h.at[pl.ds(cb(j + 3), k)], idx1)
                pltpu.make_async_copy(rows1, out_h.at[pl.ds(cb(j + 1), k)], os1).wait()
                pltpu.async_copy(table_h.at[idx1], rows1, gs1)
                return 0

            n_grp = (n_ch - 1) // 2
            lax.fori_loop(0, n_grp - 1, grp, 0)
            jl = 2 * n_grp - 2
            pltpu.make_async_copy(table_h.at[idx0], rows0, gs0).wait()
            pltpu.async_copy(rows0, out_h.at[pl.ds(cb(jl), k)], os0)
            pltpu.make_async_copy(table_h.at[idx1], rows1, gs1).wait()
            pltpu.async_copy(rows1, out_h.at[pl.ds(cb(jl + 1), k)], os1)
            pltpu.sync_copy(src_h.at[pl.ds(cb(n_ch - 1), k)], idx0)
            pltpu.make_async_copy(rows0, out_h.at[pl.ds(cb(jl), k)], os0).wait()
            pltpu.async_copy(table_h.at[idx0], rows0, gs0)
            pltpu.make_async_copy(table_h.at[idx0], rows0, gs0).wait()
            pltpu.async_copy(rows0, out_h.at[pl.ds(cb(n_ch - 1), k)], os0)
            pltpu.make_async_copy(rows0, out_h.at[pl.ds(cb(n_ch - 1), k)], os0).wait()
            pltpu.make_async_copy(rows1, out_h.at[pl.ds(cb(jl + 1), k)], os1).wait()

        one_pass(ia_h, oa_h)
        one_pass(ib_h, ob_h)

    return body(table, idx_a, idx_b)


def _translate_loop(idx_v, idx2_v, k, half, core, s, dspread=984):
    def tr(i, _):
        v = idx_v[pl.ds(i * LANES, LANES)]
        li = v - core * half
        ok = (li >= 0) & (li < half)
        dummy = half + ((i * LANES + s * 64) % dspread) + lax.iota(jnp.int32, LANES)
        idx2_v[pl.ds(i * LANES, LANES)] = jnp.where(ok, li, dummy)
        return 0

    lax.fori_loop(0, k // LANES, tr, 0)


def _sc_segsum_entity(res, head, zeros64):
    e, c = res.shape
    ch = c // 2
    n_ent = 50000
    half = n_ent // NC
    dsp = 64
    hpad = half + dsp
    k = 400
    per_s = e // NS
    n_ch = per_s // k
    assert per_s % k == 0
    zk = 1000

    @functools.partial(
        pl.kernel,
        mesh=plsc.VectorSubcoreMesh(**_MESH),
        compiler_params=pltpu.CompilerParams(use_tc_tiling_on_sc=False),
        out_type=jax.ShapeDtypeStruct((n_ent, c), jnp.float32),
        scratch_types=[
            pltpu.VMEM((k,), jnp.int32),
            pltpu.VMEM((k,), jnp.int32),
            pltpu.VMEM((k,), jnp.int32),
            pltpu.VMEM((k,), jnp.int32),
            pltpu.VMEM((k, ch), jnp.float32),
            pltpu.VMEM((k, ch), jnp.float32),
            pltpu.VMEM_SHARED((hpad, ch), jnp.float32),
            pltpu.SemaphoreType.DMA,
            pltpu.SemaphoreType.DMA,
            pltpu.SemaphoreType.DMA,
            pltpu.SemaphoreType.DMA,
        ],
    )
    def body(res_h, head_h, zeros_h, out_h,
             idx0, idx1, ix0, ix1, rows0, rows1, acc, rs0, rs1, ss0, ss1):
        core = lax.axis_index("c")
        s = lax.axis_index("s")

        def col_pass(coff):
            def zstep(i, _):
                j = s + i * NS
                @pl.when(j < half // zk)
                def _():
                    pltpu.sync_copy(zeros_h.at[:, pl.ds(0, ch)],
                                    acc.at[pl.ds(j * zk, zk)])
                return 0
            lax.fori_loop(0, (half // zk + NS - 1) // NS, zstep, 0)
            plsc.subcore_barrier()

            def cb(j):
                return s * per_s + j * k

            def load(j, idx, ix, rows, rsem):
                pltpu.sync_copy(head_h.at[pl.ds(cb(j), k)], idx)
                _translate_loop(idx, ix, k, half, core, s, dspread=dsp - 16)
                pltpu.async_copy(res_h.at[pl.ds(cb(j), k), pl.ds(coff, ch)],
                                 rows, rsem)

            load(0, idx0, ix0, rows0, rs0)
            load(1, idx1, ix1, rows1, rs1)

            def grp(g, _):
                j = 2 * g
                pltpu.make_async_copy(
                    res_h.at[pl.ds(cb(j), k), pl.ds(coff, ch)], rows0, rs0).wait()
                pltpu.async_copy(rows0, acc.at[ix0], ss0, add=True)
                pltpu.make_async_copy(
                    res_h.at[pl.ds(cb(j + 1), k), pl.ds(coff, ch)], rows1, rs1).wait()
                pltpu.async_copy(rows1, acc.at[ix1], ss1, add=True)
                pltpu.sync_copy(head_h.at[pl.ds(cb(j + 2), k)], idx0)
                pltpu.make_async_copy(rows0, acc.at[ix0], ss0).wait()
                _translate_loop(idx0, ix0, k, half, core, s, dspread=dsp - 16)
                pltpu.async_copy(res_h.at[pl.ds(cb(j + 2), k), pl.ds(coff, ch)],
                                 rows0, rs0)
                pltpu.sync_copy(head_h.at[pl.ds(cb(j + 3), k)], idx1)
                pltpu.make_async_copy(rows1, acc.at[ix1], ss1).wait()
                _translate_loop(idx1, ix1, k, half, core, s, dspread=dsp - 16)
                pltpu.async_copy(res_h.at[pl.ds(cb(j + 3), k), pl.ds(coff, ch)],
                                 rows1, rs1)
                return 0

            n_grp = (n_ch - 1) // 2
            lax.fori_loop(0, n_grp - 1, grp, 0)
            jl = 2 * n_grp - 2
            pltpu.make_async_copy(
                res_h.at[pl.ds(cb(jl), k), pl.ds(coff, ch)], rows0, rs0).wait()
            pltpu.async_copy(rows0, acc.at[ix0], ss0, add=True)
            pltpu.make_async_copy(
                res_h.at[pl.ds(cb(jl + 1), k), pl.ds(coff, ch)], rows1, rs1).wait()
            pltpu.async_copy(rows1, acc.at[ix1], ss1, add=True)
            pltpu.sync_copy(head_h.at[pl.ds(cb(n_ch - 1), k)], idx0)
            pltpu.make_async_copy(rows0, acc.at[ix0], ss0).wait()
            _translate_loop(idx0, ix0, k, half, core, s, dspread=dsp - 16)
            pltpu.async_copy(res_h.at[pl.ds(cb(n_ch - 1), k), pl.ds(coff, ch)],
                             rows0, rs0)
            pltpu.make_async_copy(
                res_h.at[pl.ds(cb(n_ch - 1), k), pl.ds(coff, ch)], rows0, rs0).wait()
            pltpu.async_copy(rows0, acc.at[ix0], ss0, add=True)
            pltpu.make_async_copy(rows0, acc.at[ix0], ss0).wait()
            pltpu.make_async_copy(rows1, acc.at[ix1], ss1).wait()
            plsc.subcore_barrier()

            def ostep(i, _):
                j = s + i * NS
                @pl.when(j < half // zk)
                def _():
                    pltpu.sync_copy(
                        acc.at[pl.ds(j * zk, zk)],
                        out_h.at[pl.ds(core * half + j * zk, zk), pl.ds(coff, ch)])
                return 0
            lax.fori_loop(0, (half // zk + NS - 1) // NS, ostep, 0)
            plsc.subcore_barrier()

        col_pass(0)
        col_pass(ch)

    return body(res, head, zeros64)


def _sc_counts(head, zeros16):
    e = head.shape[0]
    n_ent = 50000
    half = n_ent // NC
    hpad = half + 1000
    k = 2000
    per_s = e // NS
    n_ch = per_s // k
    zk = 1000

    @functools.partial(
        pl.kernel,
        mesh=plsc.VectorSubcoreMesh(**_MESH),
        compiler_params=pltpu.CompilerParams(use_tc_tiling_on_sc=False),
        out_type=jax.ShapeDtypeStruct((n_ent, 16), jnp.float32),
        scratch_types=[
            pltpu.VMEM((k,), jnp.int32),
            pltpu.VMEM((k,), jnp.int32),
            pltpu.VMEM((k, 16), jnp.float32),
            pltpu.VMEM_SHARED((hpad, 16), jnp.float32),
        ],
    )
    def body(head_h, zeros_h, out_h, idx_v, idx2_v, ones_v, acc):
        core = lax.axis_index("c")
        s = lax.axis_index("s")

        def fill(r, _):
            ones_v[r, pl.ds(0, 16)] = jnp.full((16,), 1.0, jnp.float32)
            return 0
        lax.fori_loop(0, k, fill, 0)

        def zstep(i, _):
            j = s + i * NS
            @pl.when(j < hpad // zk)
            def _():
                pltpu.sync_copy(zeros_h, acc.at[pl.ds(j * zk, zk)])
            return 0
        lax.fori_loop(0, (hpad // zk + NS - 1) // NS, zstep, 0)
        plsc.subcore_barrier()

        def step(j, _):
            base = s * per_s + j * k
            pltpu.sync_copy(head_h.at[pl.ds(base, k)], idx_v)
            _translate_loop(idx_v, idx2_v, k, half, core, s)
            pltpu.sync_copy(ones_v, acc.at[idx2_v], add=True)
            return 0
        lax.fori_loop(0, n_ch, step, 0)
        plsc.subcore_barrier()

        def ostep(i, _):
            j = s + i * NS
            @pl.when(j < half // zk)
            def _():
                pltpu.sync_copy(acc.at[pl.ds(j * zk, zk)],
                                out_h.at[pl.ds(core * half + j * zk, zk)])
            return 0
        lax.fori_loop(0, (half // zk + NS - 1) // NS, ostep, 0)

    return body(head, zeros16)


def _sc_item_agg(user_tab, row_g, col_s, zeros64):
    nnzp = row_g.shape[0]
    c = user_tab.shape[1]
    n_items = 20000
    ipad = n_items + 128
    k = 256
    per_w = nnzp // (NC * NS)
    n_ch = per_w // k
    assert per_w % k == 0 and n_ch % 2 == 0
    zk = 1000

    @functools.partial(
        pl.kernel,
        mesh=plsc.VectorSubcoreMesh(**_MESH),
        compiler_params=pltpu.CompilerParams(use_tc_tiling_on_sc=False),
        out_type=jax.ShapeDtypeStruct((NC, n_items, c), jnp.float32),
        scratch_types=[
            pltpu.VMEM((k,), jnp.int32),
            pltpu.VMEM((k,), jnp.int32),
            pltpu.VMEM((k,), jnp.int32),
            pltpu.VMEM((k,), jnp.int32),
            pltpu.VMEM((k, c), jnp.float32),
            pltpu.VMEM((k, c), jnp.float32),
            pltpu.VMEM_SHARED((ipad, c), jnp.float32),
            pltpu.SemaphoreType.DMA,
            pltpu.SemaphoreType.DMA,
            pltpu.SemaphoreType.DMA,
            pltpu.SemaphoreType.DMA,
        ],
    )
    def body(tab_h, rg_h, cs_h, zeros_h, out_h,
             ridx0, ridx1, cidx0, cidx1, rows0, rows1, acc, gs0, gs1, ss0, ss1):
        core = lax.axis_index("c")
        s = lax.axis_index("s")

        def zstep(i, _):
            j = s + i * NS
            @pl.when(j < n_items // zk)
            def _():
                pltpu.sync_copy(zeros_h, acc.at[pl.ds(j * zk, zk)])
            return 0
        lax.fori_loop(0, (n_items // zk + NS - 1) // NS, zstep, 0)
        plsc.subcore_barrier()

        def cb(j):
            return (core * NS + s) * per_w + j * k

        pltpu.sync_copy(rg_h.at[pl.ds(cb(0), k)], ridx0)
        pltpu.async_copy(tab_h.at[ridx0], rows0, gs0)
        pltpu.sync_copy(cs_h.at[pl.ds(cb(0), k)], cidx0)
        pltpu.sync_copy(rg_h.at[pl.ds(cb(1), k)], ridx1)
        pltpu.async_copy(tab_h.at[ridx1], rows1, gs1)
        pltpu.sync_copy(cs_h.at[pl.ds(cb(1), k)], cidx1)

        def grp(g, _):
            j = 2 * g
            pltpu.make_async_copy(tab_h.at[ridx0], rows0, gs0).wait()
            pltpu.async_copy(rows0, acc.at[cidx0], ss0, add=True)
            pltpu.make_async_copy(tab_h.at[ridx1], rows1, gs1).wait()
            pltpu.async_copy(rows1, acc.at[cidx1], ss1, add=True)
            pltpu.sync_copy(rg_h.at[pl.ds(cb(j + 2), k)], ridx0)
            pltpu.make_async_copy(rows0, acc.at[cidx0], ss0).wait()
            pltpu.async_copy(tab_h.at[ridx0], rows0, gs0)
            pltpu.sync_copy(cs_h.at[pl.ds(cb(j + 2), k)], cidx0)
            pltpu.sync_copy(rg_h.at[pl.ds(cb(j + 3), k)], ridx1)
            pltpu.make_async_copy(rows1, acc.at[cidx1], ss1).wait()
            pltpu.async_copy(tab_h.at[ridx1], rows1, gs1)
            pltpu.sync_copy(cs_h.at[pl.ds(cb(j + 3), k)], cidx1)
            return 0

        lax.fori_loop(0, n_ch // 2 - 1, grp, 0)
        pltpu.make_async_copy(tab_h.at[ridx0], rows0, gs0).wait()
        pltpu.async_copy(rows0, acc.at[cidx0], ss0, add=True)
        pltpu.make_async_copy(tab_h.at[ridx1], rows1, gs1).wait()
        pltpu.async_copy(rows1, acc.at[cidx1], ss1, add=True)
        pltpu.make_async_copy(rows0, acc.at[cidx0], ss0).wait()
        pltpu.make_async_copy(rows1, acc.at[cidx1], ss1).wait()
        plsc.subcore_barrier()

        def ostep(i, _):
            j = s + i * NS
            @pl.when(j < n_items // zk)
            def _():
                pltpu.sync_copy(acc.at[pl.ds(j * zk, zk)],
                                out_h.at[core, pl.ds(j * zk, zk)])
            return 0
        lax.fori_loop(0, (n_items // zk + NS - 1) // NS, ostep, 0)

    return body(user_tab, row_g, col_s, zeros64)


def _sc_user_agg(fusion_tab, col_g, row_s, val_p, zeros64):
    nnzp = col_g.shape[0]
    c = fusion_tab.shape[1]
    n_users = 50000
    half = n_users // NC
    hpad = half + 600
    k = 400
    per_s = nnzp // NS
    n_ch = per_s // k
    assert per_s % k == 0
    zk = 1000

    @functools.partial(
        pl.kernel,
        mesh=plsc.VectorSubcoreMesh(**_MESH),
        compiler_params=pltpu.CompilerParams(use_tc_tiling_on_sc=False, needs_layout_passes=False),
        out_type=jax.ShapeDtypeStruct((n_users, c), jnp.float32),
        scratch_types=[
            pltpu.VMEM((k,), jnp.int32),
            pltpu.VMEM((k,), jnp.int32),
            pltpu.VMEM((k,), jnp.float32),
            pltpu.VMEM((k, c), jnp.float32),
            pltpu.VMEM_SHARED((hpad, c), jnp.float32),
            pltpu.SemaphoreType.DMA,
        ],
    )
    def body(tab_h, cg_h, rs_h, val_h, zeros_h, out_h,
             idx_v, idx2_v, val_v, rows_v, acc, sem):
        core = lax.axis_index("c")
        s = lax.axis_index("s")

        def zstep(i, _):
            j = s + i * NS
            @pl.when(j < hpad // zk)
            def _():
                pltpu.sync_copy(zeros_h, acc.at[pl.ds(j * zk, zk)])
            return 0
        lax.fori_loop(0, (hpad // zk + NS - 1) // NS, zstep, 0)
        plsc.subcore_barrier()

        def step(j, _):
            base = s * per_s + j * k
            pltpu.sync_copy(cg_h.at[pl.ds(base, k)], idx_v)
            pltpu.async_copy(tab_h.at[idx_v], rows_v, sem).wait()
            pltpu.sync_copy(val_h.at[pl.ds(base, k)], val_v)

            def scale(r, _):
                sv = plsc.load_gather(val_v, [jnp.zeros((16,), jnp.int32) + r])
                for q in range(4):
                    rows_v[r, pl.ds(q * 16, 16)] = rows_v[r, pl.ds(q * 16, 16)] * sv
                return 0
            lax.fori_loop(0, k, scale, 0)

            pltpu.sync_copy(rs_h.at[pl.ds(base, k)], idx_v)
            _translate_loop(idx_v, idx2_v, k, half, core, s, dspread=584)
            pltpu.sync_copy(rows_v, acc.at[idx2_v], add=True)
            return 0
        lax.fori_loop(0, n_ch, step, 0)
        plsc.subcore_barrier()

        def ostep(i, _):
            j = s + i * NS
            @pl.when(j < half // zk)
            def _():
                pltpu.sync_copy(acc.at[pl.ds(j * zk, zk)],
                                out_h.at[pl.ds(core * half + j * zk, zk)])
            return 0
        lax.fori_loop(0, (half // zk + NS - 1) // NS, ostep, 0)

    return body(fusion_tab, col_g, row_s, val_p, zeros64)


def _l2norm(x):
    n = jnp.maximum(jnp.sqrt(jnp.sum(x * x, axis=-1, keepdims=True)), 1e-12)
    return x / n


def kernel(user_emb, entity_emb, item_emb_cf, relation_weight, gate1_w, gate2_w,
           mat_val, edge_index, edge_type, mat_row, mat_col):
    n_entities = entity_emb.shape[0]
    n_users = user_emb.shape[0]
    n_items = item_emb_cf.shape[0]
    c = entity_emb.shape[1]
    nnz = mat_row.shape[0]
    head = edge_index[0]
    tail = edge_index[1]

    rw_pad = jnp.zeros((16, c), jnp.float32).at[: relation_weight.shape[0]].set(relation_weight)
    zeros64 = jnp.zeros((1000, c), jnp.float32)
    zeros16 = jnp.zeros((1000, 16), jnp.float32)

    nnzp = 409600
    npad = nnzp - nnz
    ar = jnp.arange(npad, dtype=jnp.int32)
    row_g = jnp.concatenate([mat_row, ar % n_users])
    col_s = jnp.concatenate([mat_col, n_items + ar % 128])
    col_g = jnp.concatenate([mat_col, ar % n_items])
    row_s = jnp.concatenate([mat_row, jnp.full((npad,), n_users, jnp.int32)])
    val_p = jnp.concatenate([mat_val, jnp.zeros((npad,), jnp.float32)])

    cnt16 = _sc_counts(head, zeros16)
    inv_cnt = 1.0 / jnp.maximum(cnt16[:, 0], 1.0)

    e_res, u_res, i_res = entity_emb, user_emb, item_emb_cf
    cur_e, cur_u, cur_i = entity_emb, user_emb, item_emb_cf
    n_hops = gate1_w.shape[0]
    for hop in range(n_hops):
        head_emb, tail_emb = _sc_gather2(cur_e, head, tail)
        res = _edge_transform(head_emb, tail_emb, edge_type, rw_pad)
        sums = _sc_segsum_entity(res, head, zeros64)
        entity_agg = sums * inv_cnt[:, None]
        item_parts = _sc_item_agg(cur_u, row_g, col_s, zeros64)
        item_agg_cf = item_parts[0] + item_parts[1]
        item_emb_kg = cur_e[:n_items]
        gi = jax.nn.sigmoid(cur_i @ gate1_w[hop].T + item_emb_kg @ gate2_w[hop].T)
        item_fusion = gi * cur_i + (1.0 - gi) * item_emb_kg
        user_agg = _sc_user_agg(item_fusion, col_g, row_s, val_p, zeros64)
        cur_e = _l2norm(entity_agg)
        cur_u = _l2norm(user_agg)
        cur_i = _l2norm(item_agg_cf)
        e_res = e_res + cur_e
        u_res = u_res + cur_u
        i_res = i_res + cur_i
    return (e_res, u_res, i_res)


# speedup vs baseline: 1.7764x; 1.0090x over previous
import functools

import jax
import jax.numpy as jnp
from jax import lax
from jax.experimental import pallas as pl
from jax.experimental.pallas import tpu as pltpu
from jax.experimental.pallas import tpu_sc as plsc

EPS = 1e-10
MAX_NORM = 1.0 - 1e-5

NC = 2
NS = 16
LANES = 16

_MESH = dict(core_axis_name="c", subcore_axis_name="s", num_cores=NC,
             num_subcores=NS)


def _rownorm(x):
    return jnp.sqrt(jnp.sum(x * x, axis=-1, keepdims=True))


def _project(x):
    norm = jnp.maximum(_rownorm(x), EPS)
    scale = jnp.where(norm > MAX_NORM, MAX_NORM / norm, 1.0)
    return x * scale


def _mobius_add(x, y):
    x2 = jnp.sum(x * x, axis=-1, keepdims=True)
    y2 = jnp.sum(y * y, axis=-1, keepdims=True)
    xy = jnp.sum(x * y, axis=-1, keepdims=True)
    num = (1.0 + 2.0 * xy + y2) * x + (1.0 - x2) * y
    den = jnp.maximum(1.0 + 2.0 * xy + x2 * y2, EPS)
    return num / den


def _edge_coeffs(h2, t2, r2, d_ht, d_hr, d_tr):
    un_h = jnp.maximum(jnp.sqrt(h2), EPS)
    f = jnp.tanh(un_h) / un_h
    nh = jnp.sqrt(f * f * h2)
    f = f * jnp.where(nh > MAX_NORM, MAX_NORM / nh, 1.0)
    hh2 = f * f * h2
    two_over_lam = jnp.maximum(1.0 - hh2, EPS)
    hl = 1.0 / two_over_lam

    def sec(v2):
        un = jnp.maximum(jnp.sqrt(v2), EPS)
        return jnp.tanh(hl * un) / un

    def mobius(x2, y2, xy):
        den = jnp.maximum(1.0 + 2.0 * xy + x2 * y2, EPS)
        inv = 1.0 / den
        return (1.0 + 2.0 * xy + y2) * inv, (1.0 - x2) * inv

    def proj2(n2):
        n = jnp.sqrt(n2)
        return jnp.where(n > MAX_NORM, MAX_NORM / n, 1.0)

    g_t = sec(t2)
    st2 = g_t * g_t * t2
    A, B = mobius(hh2, st2, f * g_t * d_ht)
    a1, b1 = A * f, B * g_t
    ht2 = a1 * a1 * h2 + b1 * b1 * t2 + 2.0 * a1 * b1 * d_ht
    s = proj2(ht2)
    a1, b1, ht2 = a1 * s, b1 * s, ht2 * s * s

    g_r = sec(r2)
    sr2 = g_r * g_r * r2
    A, B = mobius(hh2, sr2, f * g_r * d_hr)
    a2, c2 = A * f, B * g_r
    hr2 = a2 * a2 * h2 + c2 * c2 * r2 + 2.0 * a2 * c2 * d_hr
    s = proj2(hr2)
    a2, c2, hr2 = a2 * s, c2 * s, hr2 * s * s

    xy = a1 * a2 * h2 + a1 * c2 * d_hr + b1 * a2 * d_ht + b1 * c2 * d_tr
    A, B = mobius(ht2, hr2, xy)
    al, be, ga = A * a1 + B * a2, A * b1, B * c2
    m2 = (al * al * h2 + be * be * t2 + ga * ga * r2
          + 2.0 * (al * be * d_ht + al * ga * d_hr + be * ga * d_tr))
    s = proj2(m2)
    al, be, ga, m2 = al * s, be * s, ga * s, m2 * s * s

    xy = -f * (al * h2 + be * d_ht + ga * d_hr)
    A, B = mobius(hh2, m2, xy)
    p, q, w = B * al - A * f, B * be, B * ga
    sub2 = (p * p * h2 + q * q * t2 + w * w * r2
            + 2.0 * (p * q * d_ht + p * w * d_hr + q * w * d_tr))

    sn = jnp.maximum(jnp.sqrt(sub2), EPS)
    snc = jnp.minimum(sn, MAX_NORM)
    artanh = 0.5 * (jnp.log1p(snc) - jnp.log1p(-snc))
    k0 = two_over_lam * artanh / sn
    return k0 * p, k0 * q, k0 * w


def _edge_math_block(head_emb, tail_emb, rel_emb):
    dot = lambda x, y: jnp.sum(x * y, axis=-1, keepdims=True)
    z = jnp.zeros_like(dot(head_emb, head_emb))
    g = jnp.concatenate(
        [dot(head_emb, head_emb), dot(tail_emb, tail_emb),
         dot(rel_emb, rel_emb), dot(head_emb, tail_emb),
         dot(head_emb, rel_emb), dot(tail_emb, rel_emb), z, z], axis=1)
    gt = g.T
    P, Q, W = _edge_coeffs(gt[0:1], gt[1:2], gt[2:3], gt[3:4], gt[4:5], gt[5:6])
    pqw = jnp.concatenate([P, Q, W, P, P, P, P, P], axis=0).T
    return (pqw[:, 0:1] * head_emb + pqw[:, 1:2] * tail_emb
            + pqw[:, 2:3] * rel_emb)


def _edge_kernel(head_ref, tail_ref, et_ref, rw_ref, out_ref):
    rel = et_ref[0, 0, :] - 1
    oh = (rel[:, None] == lax.iota(jnp.int32, 16)[None, :]).astype(jnp.float32)
    rel_emb = jnp.dot(oh, rw_ref[...], preferred_element_type=jnp.float32)
    out_ref[...] = _edge_math_block(head_ref[...], tail_ref[...], rel_emb)


def _pick_block(e, target=2000):
    best = 8
    for b in range(8, min(e, 4096) + 1, 8):
        if e % b == 0 and abs(b - target) < abs(best - target):
            best = b
    return best


def _edge_transform(head_emb, tail_emb, edge_type, rw_pad):
    e, c = head_emb.shape
    be = _pick_block(e)
    nb = e // be
    et3 = edge_type.reshape(nb, 1, be)
    return pl.pallas_call(
        _edge_kernel,
        grid=(nb,),
        in_specs=[
            pl.BlockSpec((be, c), lambda i: (i, 0)),
            pl.BlockSpec((be, c), lambda i: (i, 0)),
            pl.BlockSpec((1, 1, be), lambda i: (i, 0, 0)),
            pl.BlockSpec((16, c), lambda i: (0, 0)),
        ],
        out_specs=pl.BlockSpec((be, c), lambda i: (i, 0)),
        out_shape=jax.ShapeDtypeStruct((e, c), jnp.float32),
    )(head_emb, tail_emb, et3, rw_pad)


def _wid():
    return lax.axis_index("s") * NC + lax.axis_index("c")


def _sc_gather2(table, idx_a, idx_b):
    e = idx_a.shape[0]
    c = table.shape[1]
    k = 1000
    per_w = e // (NC * NS)
    n_ch = per_w // k
    assert per_w % k == 0 and n_ch % 2 == 1

    @functools.partial(
        pl.kernel,
        mesh=plsc.VectorSubcoreMesh(**_MESH),
        compiler_params=pltpu.CompilerParams(use_tc_tiling_on_sc=False),
        out_type=[jax.ShapeDtypeStruct((e, c), jnp.float32),
                  jax.ShapeDtypeStruct((e, c), jnp.float32)],
        scratch_types=[
            pltpu.VMEM((k,), jnp.int32),
            pltpu.VMEM((k,), jnp.int32),
            pltpu.VMEM((k, c), jnp.float32),
            pltpu.VMEM((k, c), jnp.float32),
            pltpu.SemaphoreType.DMA,
            pltpu.SemaphoreType.DMA,
            pltpu.SemaphoreType.DMA,
            pltpu.SemaphoreType.DMA,
        ],
    )
    def body(table_h, ia_h, ib_h, oa_h, ob_h,
             idx0, idx1, rows0, rows1, gs0, gs1, os0, os1):
        base0 = _wid() * per_w

        def one_pass(src_h, out_h):
            def cb(j):
                return base0 + j * k

            pltpu.sync_copy(src_h.at[pl.ds(cb(0), k)], idx0)
            pltpu.async_copy(table_h.at[idx0], rows0, gs0)
            pltpu.sync_copy(src_h.at[pl.ds(cb(1), k)], idx1)
            pltpu.async_copy(table_h.at[idx1], rows1, gs1)

            def grp(g, _):
                j = 2 * g
                pltpu.make_async_copy(table_h.at[idx0], rows0, gs0).wait()
                pltpu.async_copy(rows0, out_h.at[pl.ds(cb(j), k)], os0)
                pltpu.make_async_copy(table_h.at[idx1], rows1, gs1).wait()
                pltpu.async_copy(rows1, out_h.at[pl.ds(cb(j + 1), k)], os1)
                pltpu.sync_copy(src_h.at[pl.ds(cb(j + 2), k)], idx0)
                pltpu.make_async_copy(rows0, out_h.at[pl.ds(cb(j), k)], os0).wait()
                pltpu.async_copy(table_h.at[idx0], rows0, gs0)
                pltpu.sync_copy(src_h.at[pl.ds(cb(j + 3), k)], idx1)
                pltpu.make_async_copy(rows1, out_h.at[pl.ds(cb(j + 1), k)], os1).wait()
                pltpu.async_copy(table_h.at[idx1], rows1, gs1)
                return 0

            n_grp = (n_ch - 1) // 2
            lax.fori_loop(0, n_grp - 1, grp, 0)
            jl = 2 * n_grp - 2
            pltpu.make_async_copy(table_h.at[idx0], rows0, gs0).wait()
            pltpu.async_copy(rows0, out_h.at[pl.ds(cb(jl), k)], os0)
            pltpu.make_async_copy(table_h.at[idx1], rows1, gs1).wait()
            pltpu.async_copy(rows1, out_h.at[pl.ds(cb(jl + 1), k)], os1)
            pltpu.sync_copy(src_h.at[pl.ds(cb(n_ch - 1), k)], idx0)
            pltpu.make_async_copy(rows0, out_h.at[pl.ds(cb(jl), k)], os0).wait()
            pltpu.async_copy(table_h.at[idx0], rows0, gs0)
            pltpu.make_async_copy(table_h.at[idx0], rows0, gs0).wait()
            pltpu.async_copy(rows0, out_h.at[pl.ds(cb(n_ch - 1), k)], os0)
            pltpu.make_async_copy(rows0, out_h.at[pl.ds(cb(n_ch - 1), k)], os0).wait()
            pltpu.make_async_copy(rows1, out_h.at[pl.ds(cb(jl + 1), k)], os1).wait()

        one_pass(ia_h, oa_h)
        one_pass(ib_h, ob_h)

    return body(table, idx_a, idx_b)


def _translate_loop(idx_v, idx2_v, k, half, core, s, dspread=984):
    def tr(i, _):
        v = idx_v[pl.ds(i * LANES, LANES)]
        li = v - core * half
        ok = (li >= 0) & (li < half)
        dummy = half + ((i * LANES + s * 64) % dspread) + lax.iota(jnp.int32, LANES)
        idx2_v[pl.ds(i * LANES, LANES)] = jnp.where(ok, li, dummy)
        return 0

    lax.fori_loop(0, k // LANES, tr, 0)


def _sc_segsum_entity(res, head, zeros64):
    e, c = res.shape
    ch = c // 2
    n_ent = 50000
    half = n_ent // NC
    dsp = 64
    hpad = half + dsp
    k = 400
    per_s = e // NS
    n_ch = per_s // k
    assert per_s % k == 0
    zk = 1000

    @functools.partial(
        pl.kernel,
        mesh=plsc.VectorSubcoreMesh(**_MESH),
        compiler_params=pltpu.CompilerParams(use_tc_tiling_on_sc=False),
        out_type=jax.ShapeDtypeStruct((n_ent, c), jnp.float32),
        scratch_types=[
            pltpu.VMEM((k,), jnp.int32),
            pltpu.VMEM((k,), jnp.int32),
            pltpu.VMEM((k,), jnp.int32),
            pltpu.VMEM((k,), jnp.int32),
            pltpu.VMEM((k, ch), jnp.float32),
            pltpu.VMEM((k, ch), jnp.float32),
            pltpu.VMEM_SHARED((hpad, ch), jnp.float32),
            pltpu.SemaphoreType.DMA,
            pltpu.SemaphoreType.DMA,
            pltpu.SemaphoreType.DMA,
            pltpu.SemaphoreType.DMA,
        ],
    )
    def body(res_h, head_h, zeros_h, out_h,
             idx0, idx1, ix0, ix1, rows0, rows1, acc, rs0, rs1, ss0, ss1):
        core = lax.axis_index("c")
        s = lax.axis_index("s")

        def col_pass(coff):
            def zstep(i, _):
                j = s + i * NS
                @pl.when(j < half // zk)
                def _():
                    pltpu.sync_copy(zeros_h.at[:, pl.ds(0, ch)],
                                    acc.at[pl.ds(j * zk, zk)])
                return 0
            lax.fori_loop(0, (half // zk + NS - 1) // NS, zstep, 0)
            plsc.subcore_barrier()

            def cb(j):
                return s * per_s + j * k

            def load(j, idx, ix, rows, rsem):
                pltpu.sync_copy(head_h.at[pl.ds(cb(j), k)], idx)
                _translate_loop(idx, ix, k, half, core, s, dspread=dsp - 16)
                pltpu.async_copy(res_h.at[pl.ds(cb(j), k), pl.ds(coff, ch)],
                                 rows, rsem)

            load(0, idx0, ix0, rows0, rs0)
            load(1, idx1, ix1, rows1, rs1)

            def grp(g, _):
                j = 2 * g
                pltpu.make_async_copy(
                    res_h.at[pl.ds(cb(j), k), pl.ds(coff, ch)], rows0, rs0).wait()
                pltpu.async_copy(rows0, acc.at[ix0], ss0, add=True)
                pltpu.make_async_copy(
                    res_h.at[pl.ds(cb(j + 1), k), pl.ds(coff, ch)], rows1, rs1).wait()
                pltpu.async_copy(rows1, acc.at[ix1], ss1, add=True)
                pltpu.sync_copy(head_h.at[pl.ds(cb(j + 2), k)], idx0)
                pltpu.make_async_copy(rows0, acc.at[ix0], ss0).wait()
                _translate_loop(idx0, ix0, k, half, core, s, dspread=dsp - 16)
                pltpu.async_copy(res_h.at[pl.ds(cb(j + 2), k), pl.ds(coff, ch)],
                                 rows0, rs0)
                pltpu.sync_copy(head_h.at[pl.ds(cb(j + 3), k)], idx1)
                pltpu.make_async_copy(rows1, acc.at[ix1], ss1).wait()
                _translate_loop(idx1, ix1, k, half, core, s, dspread=dsp - 16)
                pltpu.async_copy(res_h.at[pl.ds(cb(j + 3), k), pl.ds(coff, ch)],
                                 rows1, rs1)
                return 0

            n_grp = (n_ch - 1) // 2
            lax.fori_loop(0, n_grp - 1, grp, 0)
            jl = 2 * n_grp - 2
            pltpu.make_async_copy(
                res_h.at[pl.ds(cb(jl), k), pl.ds(coff, ch)], rows0, rs0).wait()
            pltpu.async_copy(rows0, acc.at[ix0], ss0, add=True)
            pltpu.make_async_copy(
                res_h.at[pl.ds(cb(jl + 1), k), pl.ds(coff, ch)], rows1, rs1).wait()
            pltpu.async_copy(rows1, acc.at[ix1], ss1, add=True)
            pltpu.sync_copy(head_h.at[pl.ds(cb(n_ch - 1), k)], idx0)
            pltpu.make_async_copy(rows0, acc.at[ix0], ss0).wait()
            _translate_loop(idx0, ix0, k, half, core, s, dspread=dsp - 16)
            pltpu.async_copy(res_h.at[pl.ds(cb(n_ch - 1), k), pl.ds(coff, ch)],
                             rows0, rs0)
            pltpu.make_async_copy(
                res_h.at[pl.ds(cb(n_ch - 1), k), pl.ds(coff, ch)], rows0, rs0).wait()
            pltpu.async_copy(rows0, acc.at[ix0], ss0, add=True)
            pltpu.make_async_copy(rows0, acc.at[ix0], ss0).wait()
            pltpu.make_async_copy(rows1, acc.at[ix1], ss1).wait()
            plsc.subcore_barrier()

            def ostep(i, _):
                j = s + i * NS
                @pl.when(j < half // zk)
                def _():
                    pltpu.sync_copy(
                        acc.at[pl.ds(j * zk, zk)],
                        out_h.at[pl.ds(core * half + j * zk, zk), pl.ds(coff, ch)])
                return 0
            lax.fori_loop(0, (half // zk + NS - 1) // NS, ostep, 0)
            plsc.subcore_barrier()

        col_pass(0)
        col_pass(ch)

    return body(res, head, zeros64)


def _sc_counts(head, zeros16):
    e = head.shape[0]
    n_ent = 50000
    half = n_ent // NC
    hpad = half + 1000
    k = 2000
    per_s = e // NS
    n_ch = per_s // k
    zk = 1000

    @functools.partial(
        pl.kernel,
        mesh=plsc.VectorSubcoreMesh(**_MESH),
        compiler_params=pltpu.CompilerParams(use_tc_tiling_on_sc=False),
        out_type=jax.ShapeDtypeStruct((n_ent, 16), jnp.float32),
        scratch_types=[
            pltpu.VMEM((k,), jnp.int32),
            pltpu.VMEM((k,), jnp.int32),
            pltpu.VMEM((k, 16), jnp.float32),
            pltpu.VMEM_SHARED((hpad, 16), jnp.float32),
        ],
    )
    def body(head_h, zeros_h, out_h, idx_v, idx2_v, ones_v, acc):
        core = lax.axis_index("c")
        s = lax.axis_index("s")

        def fill(r, _):
            ones_v[r, pl.ds(0, 16)] = jnp.full((16,), 1.0, jnp.float32)
            return 0
        lax.fori_loop(0, k, fill, 0)

        def zstep(i, _):
            j = s + i * NS
            @pl.when(j < hpad // zk)
            def _():
                pltpu.sync_copy(zeros_h, acc.at[pl.ds(j * zk, zk)])
            return 0
        lax.fori_loop(0, (hpad // zk + NS - 1) // NS, zstep, 0)
        plsc.subcore_barrier()

        def step(j, _):
            base = s * per_s + j * k
            pltpu.sync_copy(head_h.at[pl.ds(base, k)], idx_v)
            _translate_loop(idx_v, idx2_v, k, half, core, s)
            pltpu.sync_copy(ones_v, acc.at[idx2_v], add=True)
            return 0
        lax.fori_loop(0, n_ch, step, 0)
        plsc.subcore_barrier()

        def ostep(i, _):
            j = s + i * NS
            @pl.when(j < half // zk)
            def _():
                pltpu.sync_copy(acc.at[pl.ds(j * zk, zk)],
                                out_h.at[pl.ds(core * half + j * zk, zk)])
            return 0
        lax.fori_loop(0, (half // zk + NS - 1) // NS, ostep, 0)

    return body(head, zeros16)


def _sc_item_agg(user_tab, row_g, col_s, zeros64):
    nnzp = row_g.shape[0]
    c = user_tab.shape[1]
    n_items = 20000
    ipad = n_items + 128
    k = 256
    per_w = nnzp // (NC * NS)
    n_ch = per_w // k
    assert per_w % k == 0 and n_ch % 2 == 0
    zk = 1000

    @functools.partial(
        pl.kernel,
        mesh=plsc.VectorSubcoreMesh(**_MESH),
        compiler_params=pltpu.CompilerParams(use_tc_tiling_on_sc=False),
        out_type=jax.ShapeDtypeStruct((NC, n_items, c), jnp.float32),
        scratch_types=[
            pltpu.VMEM((k,), jnp.int32),
            pltpu.VMEM((k,), jnp.int32),
            pltpu.VMEM((k,), jnp.int32),
            pltpu.VMEM((k,), jnp.int32),
            pltpu.VMEM((k, c), jnp.float32),
            pltpu.VMEM((k, c), jnp.float32),
            pltpu.VMEM_SHARED((ipad, c), jnp.float32),
            pltpu.SemaphoreType.DMA,
            pltpu.SemaphoreType.DMA,
            pltpu.SemaphoreType.DMA,
            pltpu.SemaphoreType.DMA,
        ],
    )
    def body(tab_h, rg_h, cs_h, zeros_h, out_h,
             ridx0, ridx1, cidx0, cidx1, rows0, rows1, acc, gs0, gs1, ss0, ss1):
        core = lax.axis_index("c")
        s = lax.axis_index("s")

        def zstep(i, _):
            j = s + i * NS
            @pl.when(j < n_items // zk)
            def _():
                pltpu.sync_copy(zeros_h, acc.at[pl.ds(j * zk, zk)])
            return 0
        lax.fori_loop(0, (n_items // zk + NS - 1) // NS, zstep, 0)
        plsc.subcore_barrier()

        def cb(j):
            return (core * NS + s) * per_w + j * k

        pltpu.sync_copy(rg_h.at[pl.ds(cb(0), k)], ridx0)
        pltpu.async_copy(tab_h.at[ridx0], rows0, gs0)
        pltpu.sync_copy(cs_h.at[pl.ds(cb(0), k)], cidx0)
        pltpu.sync_copy(rg_h.at[pl.ds(cb(1), k)], ridx1)
        pltpu.async_copy(tab_h.at[ridx1], rows1, gs1)
        pltpu.sync_copy(cs_h.at[pl.ds(cb(1), k)], cidx1)

        def grp(g, _):
            j = 2 * g
            pltpu.make_async_copy(tab_h.at[ridx0], rows0, gs0).wait()
            pltpu.async_copy(rows0, acc.at[cidx0], ss0, add=True)
            pltpu.make_async_copy(tab_h.at[ridx1], rows1, gs1).wait()
            pltpu.async_copy(rows1, acc.at[cidx1], ss1, add=True)
            pltpu.sync_copy(rg_h.at[pl.ds(cb(j + 2), k)], ridx0)
            pltpu.make_async_copy(rows0, acc.at[cidx0], ss0).wait()
            pltpu.async_copy(tab_h.at[ridx0], rows0, gs0)
            pltpu.sync_copy(cs_h.at[pl.ds(cb(j + 2), k)], cidx0)
            pltpu.sync_copy(rg_h.at[pl.ds(cb(j + 3), k)], ridx1)
            pltpu.make_async_copy(rows1, acc.at[cidx1], ss1).wait()
            pltpu.async_copy(tab_h.at[ridx1], rows1, gs1)
            pltpu.sync_copy(cs_h.at[pl.ds(cb(j + 3), k)], cidx1)
            return 0

        lax.fori_loop(0, n_ch // 2 - 1, grp, 0)
        pltpu.make_async_copy(tab_h.at[ridx0], rows0, gs0).wait()
        pltpu.async_copy(rows0, acc.at[cidx0], ss0, add=True)
        pltpu.make_async_copy(tab_h.at[ridx1], rows1, gs1).wait()
        pltpu.async_copy(rows1, acc.at[cidx1], ss1, add=True)
        pltpu.make_async_copy(rows0, acc.at[cidx0], ss0).wait()
        pltpu.make_async_copy(rows1, acc.at[cidx1], ss1).wait()
        plsc.subcore_barrier()

        def ostep(i, _):
            j = s + i * NS
            @pl.when(j < n_items // zk)
            def _():
                pltpu.sync_copy(acc.at[pl.ds(j * zk, zk)],
                                out_h.at[core, pl.ds(j * zk, zk)])
            return 0
        lax.fori_loop(0, (n_items // zk + NS - 1) // NS, ostep, 0)

    return body(user_tab, row_g, col_s, zeros64)


def _sc_user_agg(fusion_tab, col_g, row_s, val_p, zeros64):
    nnzp = col_g.shape[0]
    c = fusion_tab.shape[1]
    n_users = 50000
    half = n_users // NC
    dsp = 64
    hpad = half + dsp
    k = 160
    per_s = nnzp // NS
    n_ch = per_s // k
    assert per_s % k == 0 and n_ch % 2 == 0
    zk = 1000

    @functools.partial(
        pl.kernel,
        mesh=plsc.VectorSubcoreMesh(**_MESH),
        compiler_params=pltpu.CompilerParams(use_tc_tiling_on_sc=False, needs_layout_passes=False),
        out_type=jax.ShapeDtypeStruct((n_users, c), jnp.float32),
        scratch_types=[
            pltpu.VMEM((k,), jnp.int32),
            pltpu.VMEM((k,), jnp.int32),
            pltpu.VMEM((k,), jnp.int32),
            pltpu.VMEM((k,), jnp.int32),
            pltpu.VMEM((k,), jnp.float32),
            pltpu.VMEM((k,), jnp.float32),
            pltpu.VMEM((k, c), jnp.float32),
            pltpu.VMEM((k, c), jnp.float32),
            pltpu.VMEM_SHARED((hpad, c), jnp.float32),
            pltpu.SemaphoreType.DMA,
            pltpu.SemaphoreType.DMA,
            pltpu.SemaphoreType.DMA,
            pltpu.SemaphoreType.DMA,
        ],
    )
    def body(tab_h, cg_h, rs_h, val_h, zeros_h, out_h,
             idx0, idx1, ix0, ix1, val0, val1, rows0, rows1, acc,
             gs0, gs1, ss0, ss1):
        core = lax.axis_index("c")
        s = lax.axis_index("s")

        def zstep(i, _):
            j = s + i * NS
            @pl.when(j < half // zk)
            def _():
                pltpu.sync_copy(zeros_h, acc.at[pl.ds(j * zk, zk)])
            return 0
        lax.fori_loop(0, (half // zk + NS - 1) // NS, zstep, 0)
        plsc.subcore_barrier()

        def cb(j):
            return s * per_s + j * k

        def prep(j, idx, ix, val, rows, gsem):
            pltpu.sync_copy(cg_h.at[pl.ds(cb(j), k)], idx)
            pltpu.async_copy(tab_h.at[idx], rows, gsem)
            pltpu.sync_copy(val_h.at[pl.ds(cb(j), k)], val)
            pltpu.sync_copy(rs_h.at[pl.ds(cb(j), k)], ix)
            _translate_loop(ix, ix, k, half, core, s, dspread=dsp - 16)

        def scale(val, rows):
            def srow(r, _):
                sv = plsc.load_gather(val, [jnp.zeros((16,), jnp.int32) + r])
                for q in range(4):
                    rows[r, pl.ds(q * 16, 16)] = rows[r, pl.ds(q * 16, 16)] * sv
                return 0
            lax.fori_loop(0, k, srow, 0)

        prep(0, idx0, ix0, val0, rows0, gs0)
        prep(1, idx1, ix1, val1, rows1, gs1)

        def grp(g, _):
            j = 2 * g
            pltpu.make_async_copy(tab_h.at[idx0], rows0, gs0).wait()
            scale(val0, rows0)
            pltpu.async_copy(rows0, acc.at[ix0], ss0, add=True)
            pltpu.make_async_copy(tab_h.at[idx1], rows1, gs1).wait()
            scale(val1, rows1)
            pltpu.async_copy(rows1, acc.at[ix1], ss1, add=True)
            pltpu.make_async_copy(rows0, acc.at[ix0], ss0).wait()
            prep(j + 2, idx0, ix0, val0, rows0, gs0)
            pltpu.make_async_copy(rows1, acc.at[ix1], ss1).wait()
            prep(j + 3, idx1, ix1, val1, rows1, gs1)
            return 0

        lax.fori_loop(0, n_ch // 2 - 1, grp, 0)
        pltpu.make_async_copy(tab_h.at[idx0], rows0, gs0).wait()
        scale(val0, rows0)
        pltpu.async_copy(rows0, acc.at[ix0], ss0, add=True)
        pltpu.make_async_copy(tab_h.at[idx1], rows1, gs1).wait()
        scale(val1, rows1)
        pltpu.async_copy(rows1, acc.at[ix1], ss1, add=True)
        pltpu.make_async_copy(rows0, acc.at[ix0], ss0).wait()
        pltpu.make_async_copy(rows1, acc.at[ix1], ss1).wait()
        plsc.subcore_barrier()

        def ostep(i, _):
            j = s + i * NS
            @pl.when(j < half // zk)
            def _():
                pltpu.sync_copy(acc.at[pl.ds(j * zk, zk)],
                                out_h.at[pl.ds(core * half + j * zk, zk)])
            return 0
        lax.fori_loop(0, (half // zk + NS - 1) // NS, ostep, 0)

    return body(fusion_tab, col_g, row_s, val_p, zeros64)


def _l2norm(x):
    n = jnp.maximum(jnp.sqrt(jnp.sum(x * x, axis=-1, keepdims=True)), 1e-12)
    return x / n


def kernel(user_emb, entity_emb, item_emb_cf, relation_weight, gate1_w, gate2_w,
           mat_val, edge_index, edge_type, mat_row, mat_col):
    n_entities = entity_emb.shape[0]
    n_users = user_emb.shape[0]
    n_items = item_emb_cf.shape[0]
    c = entity_emb.shape[1]
    nnz = mat_row.shape[0]
    head = edge_index[0]
    tail = edge_index[1]

    rw_pad = jnp.zeros((16, c), jnp.float32).at[: relation_weight.shape[0]].set(relation_weight)
    zeros64 = jnp.zeros((1000, c), jnp.float32)
    zeros16 = jnp.zeros((1000, 16), jnp.float32)

    nnzp = 409600
    npad = nnzp - nnz
    ar = jnp.arange(npad, dtype=jnp.int32)
    row_g = jnp.concatenate([mat_row, ar % n_users])
    col_s = jnp.concatenate([mat_col, n_items + ar % 128])
    col_g = jnp.concatenate([mat_col, ar % n_items])
    row_s = jnp.concatenate([mat_row, jnp.full((npad,), n_users, jnp.int32)])
    val_p = jnp.concatenate([mat_val, jnp.zeros((npad,), jnp.float32)])

    cnt16 = _sc_counts(head, zeros16)
    inv_cnt = 1.0 / jnp.maximum(cnt16[:, 0], 1.0)

    e_res, u_res, i_res = entity_emb, user_emb, item_emb_cf
    cur_e, cur_u, cur_i = entity_emb, user_emb, item_emb_cf
    n_hops = gate1_w.shape[0]
    for hop in range(n_hops):
        head_emb, tail_emb = _sc_gather2(cur_e, head, tail)
        res = _edge_transform(head_emb, tail_emb, edge_type, rw_pad)
        sums = _sc_segsum_entity(res, head, zeros64)
        entity_agg = sums * inv_cnt[:, None]
        item_parts = _sc_item_agg(cur_u, row_g, col_s, zeros64)
        item_agg_cf = item_parts[0] + item_parts[1]
        item_emb_kg = cur_e[:n_items]
        gi = jax.nn.sigmoid(cur_i @ gate1_w[hop].T + item_emb_kg @ gate2_w[hop].T)
        item_fusion = gi * cur_i + (1.0 - gi) * item_emb_kg
        user_agg = _sc_user_agg(item_fusion, col_g, row_s, val_p, zeros64)
        cur_e = _l2norm(entity_agg)
        cur_u = _l2norm(user_agg)
        cur_i = _l2norm(item_agg_cf)
        e_res = e_res + cur_e
        u_res = u_res + cur_u
        i_res = i_res + cur_i
    return (e_res, u_res, i_res)
